# baseline (speedup 1.0000x reference)
import functools

import jax
import jax.numpy as jnp
from jax import lax
from jax.experimental import pallas as pl
from jax.experimental.pallas import tpu as pltpu
from jax.experimental.pallas import tpu_sc as plsc

N = 10000
E = 320000
NC = 2
NS = 16
NW = NC * NS
K = 96
NCHUNK = 106
NHALF = NCHUNK // 2
E_PT = NCHUNK * K
E_PAD = NW * E_PT
N_ACC = 10240
ROWS_PT = N_ACC // NS
PAD_W = 16

_MESH = dict(core_axis_name="c", subcore_axis_name="s")


def _make_degree_kernel():
    @functools.partial(
        pl.kernel,
        mesh=plsc.VectorSubcoreMesh(**_MESH),
        compiler_params=pltpu.CompilerParams(use_tc_tiling_on_sc=False),
        out_type=jax.ShapeDtypeStruct((NC, N_ACC, PAD_W), jnp.float32),
        scratch_types=[
            pltpu.VMEM((NCHUNK, K), jnp.int32),
            pltpu.VMEM((K, PAD_W), jnp.float32),
            pltpu.VMEM_SHARED((N_ACC, PAD_W), jnp.float32),
            pltpu.SemaphoreType.DMA,
            pltpu.SemaphoreType.DMA,
        ],
    )
    def deg_kernel(dst_hbm, ones_hbm, zeros_hbm, out_hbm, dst_v, ones_v, acc_sh,
                   s0, s1):
        cid = lax.axis_index("c")
        sid = lax.axis_index("s")
        pltpu.sync_copy(dst_hbm.at[cid, sid], dst_v)
        pltpu.sync_copy(ones_hbm, ones_v)
        sl = pl.ds(sid * ROWS_PT, ROWS_PT)
        pltpu.sync_copy(zeros_hbm.at[sl], acc_sh.at[sl])
        plsc.subcore_barrier()

        def body(i, carry):
            d0 = pltpu.async_copy(ones_v, acc_sh.at[dst_v.at[2 * i]], s0, add=True)
            d1 = pltpu.async_copy(ones_v, acc_sh.at[dst_v.at[2 * i + 1]], s1, add=True)
            d0.wait()
            d1.wait()
            return carry

        lax.fori_loop(0, NHALF, body, 0)
        plsc.subcore_barrier()
        pltpu.sync_copy(acc_sh.at[sl], out_hbm.at[cid, sl])

    return deg_kernel


def _make_spmm_kernel(D):
    @functools.partial(
        pl.kernel,
        mesh=plsc.VectorSubcoreMesh(**_MESH),
        compiler_params=pltpu.CompilerParams(use_tc_tiling_on_sc=False),
        out_type=jax.ShapeDtypeStruct((NC, N_ACC, D), jnp.float32),
        scratch_types=[
            pltpu.VMEM((NCHUNK, K), jnp.int32),
            pltpu.VMEM((NCHUNK, K), jnp.int32),
            pltpu.VMEM((K, D), jnp.float32),
            pltpu.VMEM((K, D), jnp.float32),
            pltpu.VMEM_SHARED((N_ACC, D), jnp.float32),
            pltpu.SemaphoreType.DMA,
            pltpu.SemaphoreType.DMA,
            pltpu.SemaphoreType.DMA,
            pltpu.SemaphoreType.DMA,
        ],
    )
    def spmm(g_hbm, src_hbm, dst_hbm, zeros_hbm, out_hbm,
             src_v, dst_v, rows0, rows1, acc_sh, gs0, gs1, ss0, ss1):
        cid = lax.axis_index("c")
        sid = lax.axis_index("s")
        pltpu.sync_copy(src_hbm.at[cid, sid], src_v)
        pltpu.sync_copy(dst_hbm.at[cid, sid], dst_v)
        sl = pl.ds(sid * ROWS_PT, ROWS_PT)
        pltpu.sync_copy(zeros_hbm.at[sl], acc_sh.at[sl])
        plsc.subcore_barrier()

        def body(j, carry):
            pltpu.async_copy(g_hbm.at[src_v.at[j]], rows0, gs0).wait()
            pltpu.async_copy(rows0, acc_sh.at[dst_v.at[j]], ss0, add=True).wait()
            return carry

        lax.fori_loop(0, NCHUNK, body, 0)
        plsc.subcore_barrier()
        pltpu.sync_copy(acc_sh.at[sl], out_hbm.at[cid, sl])

    return spmm


_deg_call = _make_degree_kernel()
_spmm128 = _make_spmm_kernel(128)
_spmm64 = _make_spmm_kernel(64)
_spmm16 = _make_spmm_kernel(PAD_W)


_TB = 1000
_GRID = (N // _TB,)


def _dis_block(degp):
    deg = degp[0, :, 0:1] + degp[1, :, 0:1]
    return jnp.where(deg > 0.0, lax.rsqrt(deg), 0.0)


def _t1_body(x_ref, fcW_ref, fcb_ref, c1W_ref, degp_ref, x1_ref, g1_ref):
    xb = x_ref[...]
    x1_ref[...] = jnp.maximum(
        jnp.dot(xb, fcW_ref[...], preferred_element_type=jnp.float32) + fcb_ref[...], 0.0)
    dis = _dis_block(degp_ref[...])
    g1_ref[...] = jnp.dot(xb, c1W_ref[...], preferred_element_type=jnp.float32) * dis


def _t2_body(x1_ref, acc1_ref, degp_ref, c1b_ref, fc1Wa_ref, fc1Wb_ref,
             fc1b_ref, c2Wa_ref, c2Wb_ref, x5_ref, g2_ref):
    dis = _dis_block(degp_ref[...])
    s1 = acc1_ref[0] + acc1_ref[1]
    x2 = jnp.maximum(dis * s1 + c1b_ref[...], 0.0)
    x1b = x1_ref[...]
    x5_ref[...] = jnp.maximum(
        jnp.dot(x1b, fc1Wa_ref[...], preferred_element_type=jnp.float32)
        + jnp.dot(x2, fc1Wb_ref[...], preferred_element_type=jnp.float32)
        + fc1b_ref[...], 0.0)
    g2_ref[...] = (jnp.dot(x1b, c2Wa_ref[...], preferred_element_type=jnp.float32)
                   + jnp.dot(x2, c2Wb_ref[...], preferred_element_type=jnp.float32)) * dis


def _t3_body(x5_ref, acc2_ref, degp_ref, c2b_ref, W9_ref, b9_ref, W3_ref,
             x9_ref, g3_ref):
    dis = _dis_block(degp_ref[...])
    s2 = acc2_ref[0] + acc2_ref[1]
    x6 = jnp.maximum(dis * s2 + c2b_ref[...], 0.0)
    x7 = x5_ref[...] + x6
    x9_ref[...] = jnp.dot(x7, W9_ref[...], preferred_element_type=jnp.float32) + b9_ref[...]
    g3_ref[...] = jnp.dot(x7, W3_ref[...], preferred_element_type=jnp.float32) * dis


def _t4_body(x9_ref, acc3_ref, degp_ref, c3b_ref, out_ref):
    dis = _dis_block(degp_ref[...])
    s3 = acc3_ref[0] + acc3_ref[1]
    out_ref[...] = x9_ref[...] + dis * s3 + c3b_ref[...]


def _row_spec(w):
    return pl.BlockSpec((_TB, w), lambda i: (i, 0))


def _part_spec(w):
    return pl.BlockSpec((NC, _TB, w), lambda i: (0, i, 0))


def _full_spec(shape):
    nd = len(shape)
    return pl.BlockSpec(shape, lambda i: (0,) * nd)


def _sds(shape):
    return jax.ShapeDtypeStruct(shape, jnp.float32)


_t1_call = pl.pallas_call(
    _t1_body,
    grid=_GRID,
    in_specs=[_row_spec(128), _full_spec((128, 128)), _full_spec((1, 128)),
              _full_spec((128, 128)), _part_spec(PAD_W)],
    out_specs=[_row_spec(128), _row_spec(128)],
    out_shape=[_sds((N, 128)), _sds((N, 128))],
)

_t2_call = pl.pallas_call(
    _t2_body,
    grid=_GRID,
    in_specs=[_row_spec(128), _part_spec(128), _part_spec(PAD_W),
              _full_spec((1, 128)), _full_spec((128, 64)), _full_spec((128, 64)),
              _full_spec((1, 64)), _full_spec((128, 64)), _full_spec((128, 64))],
    out_specs=[_row_spec(64), _row_spec(64)],
    out_shape=[_sds((N, 64)), _sds((N, 64))],
)

_t3_call = pl.pallas_call(
    _t3_body,
    grid=_GRID,
    in_specs=[_row_spec(64), _part_spec(64), _part_spec(PAD_W),
              _full_spec((1, 64)), _full_spec((64, PAD_W)),
              _full_spec((1, PAD_W)), _full_spec((64, PAD_W))],
    out_specs=[_row_spec(PAD_W), _row_spec(PAD_W)],
    out_shape=[_sds((N, PAD_W)), _sds((N, PAD_W))],
)

_t4_call = pl.pallas_call(
    _t4_body,
    grid=_GRID,
    in_specs=[_row_spec(PAD_W), _part_spec(PAD_W), _part_spec(PAD_W),
              _full_spec((1, PAD_W))],
    out_specs=_row_spec(PAD_W),
    out_shape=_sds((N, PAD_W)),
)


def _pad16(a):
    return jnp.pad(a, ((0, 0), (0, PAD_W - a.shape[1])))


def kernel(x, edge_index, fc_W, fc_b, conv1_W, conv1_b, fc1_W, fc1_b,
           conv2_W, conv2_b, fc2_W, fc2_b, conv3_W, conv3_b):
    npad = E_PAD - E
    pad_dst = N + (jnp.arange(npad, dtype=jnp.int32) % (N_ACC - N))
    src_p = jnp.concatenate([edge_index[0], jnp.zeros((npad,), jnp.int32)])
    dst_p = jnp.concatenate([edge_index[1], pad_dst])
    src_r = src_p.reshape(NC, NS, NCHUNK, K)
    dst_r = dst_p.reshape(NC, NS, NCHUNK, K)

    ones = jnp.ones((K, PAD_W), jnp.float32)
    zeros16 = jnp.zeros((N_ACC, PAD_W), jnp.float32)
    zeros64 = jnp.zeros((N_ACC, 64), jnp.float32)
    zeros128 = jnp.zeros((N_ACC, 128), jnp.float32)

    degp = _deg_call(dst_r, ones, zeros16)

    x1, g1 = _t1_call(x, fc_W, fc_b.reshape(1, -1), conv1_W, degp)
    acc1 = _spmm128(g1, src_r, dst_r, zeros128)

    x5, g2 = _t2_call(x1, acc1, degp, conv1_b.reshape(1, -1),
                      fc1_W[:128], fc1_W[128:], fc1_b.reshape(1, -1),
                      conv2_W[:128], conv2_W[128:])
    acc2 = _spmm64(g2, src_r, dst_r, zeros64)

    x9p, g3 = _t3_call(x5, acc2, degp, conv2_b.reshape(1, -1),
                       _pad16(fc2_W), _pad16(fc2_b.reshape(1, 1)), _pad16(conv3_W))
    acc3 = _spmm16(g3, src_r, dst_r, zeros16)

    outp = _t4_call(x9p, acc3, degp, _pad16(conv3_b.reshape(1, 1)))
    return outp[:, 0:1]


# speedup vs baseline: 1.0589x; 1.0589x over previous
import functools

import jax
import jax.numpy as jnp
from jax import lax
from jax.experimental import pallas as pl
from jax.experimental.pallas import tpu as pltpu
from jax.experimental.pallas import tpu_sc as plsc

N = 10000
E = 320000
NC = 2
NS = 16
NW = NC * NS
K = 80
NCHUNK = 128
NHALF = NCHUNK // 2
E_PT = NCHUNK * K
E_PAD = NW * E_PT
N_ACC = 10240
ROWS_PT = N_ACC // NS
PAD_W = 16

_MESH = dict(core_axis_name="c", subcore_axis_name="s")


def _make_degree_kernel():
    @functools.partial(
        pl.kernel,
        mesh=plsc.VectorSubcoreMesh(**_MESH),
        compiler_params=pltpu.CompilerParams(use_tc_tiling_on_sc=False),
        out_type=jax.ShapeDtypeStruct((NC, N_ACC, PAD_W), jnp.float32),
        scratch_types=[
            pltpu.VMEM((NCHUNK, K), jnp.int32),
            pltpu.VMEM((K, PAD_W), jnp.float32),
            pltpu.VMEM_SHARED((N_ACC, PAD_W), jnp.float32),
            pltpu.SemaphoreType.DMA,
            pltpu.SemaphoreType.DMA,
        ],
    )
    def deg_kernel(dst_hbm, ones_hbm, zeros_hbm, out_hbm, dst_v, ones_v, acc_sh,
                   s0, s1):
        cid = lax.axis_index("c")
        sid = lax.axis_index("s")
        pltpu.sync_copy(dst_hbm.at[cid, sid], dst_v)
        pltpu.sync_copy(ones_hbm, ones_v)
        sl = pl.ds(sid * ROWS_PT, ROWS_PT)
        pltpu.sync_copy(zeros_hbm.at[sl], acc_sh.at[sl])
        plsc.subcore_barrier()

        def body(i, carry):
            d0 = pltpu.async_copy(ones_v, acc_sh.at[dst_v.at[2 * i]], s0, add=True)
            d1 = pltpu.async_copy(ones_v, acc_sh.at[dst_v.at[2 * i + 1]], s1, add=True)
            d0.wait()
            d1.wait()
            return carry

        lax.fori_loop(0, NHALF, body, 0)
        plsc.subcore_barrier()
        pltpu.sync_copy(acc_sh.at[sl], out_hbm.at[cid, sl])

    return deg_kernel


def _make_spmm_kernel(D):
    @functools.partial(
        pl.kernel,
        mesh=plsc.VectorSubcoreMesh(**_MESH),
        compiler_params=pltpu.CompilerParams(use_tc_tiling_on_sc=False),
        out_type=jax.ShapeDtypeStruct((NC, N_ACC, D), jnp.float32),
        scratch_types=[
            pltpu.VMEM((NCHUNK, K), jnp.int32),
            pltpu.VMEM((NCHUNK, K), jnp.int32),
            pltpu.VMEM((K, D), jnp.float32),
            pltpu.VMEM((K, D), jnp.float32),
            pltpu.VMEM_SHARED((N_ACC, D), jnp.float32),
            pltpu.SemaphoreType.DMA,
            pltpu.SemaphoreType.DMA,
            pltpu.SemaphoreType.DMA,
            pltpu.SemaphoreType.DMA,
        ],
    )
    def spmm(g_hbm, src_hbm, dst_hbm, zeros_hbm, out_hbm,
             src_v, dst_v, rows0, rows1, acc_sh, gs0, gs1, ss0, ss1):
        cid = lax.axis_index("c")
        sid = lax.axis_index("s")
        pltpu.sync_copy(src_hbm.at[cid, sid], src_v)
        pltpu.sync_copy(dst_hbm.at[cid, sid], dst_v)
        sl = pl.ds(sid * ROWS_PT, ROWS_PT)
        pltpu.sync_copy(zeros_hbm.at[sl], acc_sh.at[sl])
        plsc.subcore_barrier()

        pltpu.async_copy(g_hbm.at[src_v.at[0]], rows0, gs0)
        pltpu.async_copy(g_hbm.at[src_v.at[1]], rows1, gs1)

        def body(i, carry):
            j0 = 2 * i
            j1 = 2 * i + 1
            pltpu.make_async_copy(g_hbm.at[src_v.at[j0]], rows0, gs0).wait()
            pltpu.async_copy(rows0, acc_sh.at[dst_v.at[j0]], ss0, add=True).wait()

            @pl.when(j0 + 2 < NCHUNK)
            def _():
                pltpu.async_copy(g_hbm.at[src_v.at[j0 + 2]], rows0, gs0)

            pltpu.make_async_copy(g_hbm.at[src_v.at[j1]], rows1, gs1).wait()
            pltpu.async_copy(rows1, acc_sh.at[dst_v.at[j1]], ss1, add=True).wait()

            @pl.when(j1 + 2 < NCHUNK)
            def _():
                pltpu.async_copy(g_hbm.at[src_v.at[j1 + 2]], rows1, gs1)

            return carry

        lax.fori_loop(0, NHALF, body, 0)
        plsc.subcore_barrier()
        pltpu.sync_copy(acc_sh.at[sl], out_hbm.at[cid, sl])

    return spmm


_deg_call = _make_degree_kernel()
_spmm128 = _make_spmm_kernel(128)
_spmm64 = _make_spmm_kernel(64)
_spmm16 = _make_spmm_kernel(PAD_W)


_TB = 1000
_GRID = (N // _TB,)


def _dis_block(degp):
    deg = degp[0, :, 0:1] + degp[1, :, 0:1]
    return jnp.where(deg > 0.0, lax.rsqrt(deg), 0.0)


def _t1_body(x_ref, fcW_ref, fcb_ref, c1W_ref, degp_ref, x1_ref, g1_ref):
    xb = x_ref[...]
    x1_ref[...] = jnp.maximum(
        jnp.dot(xb, fcW_ref[...], preferred_element_type=jnp.float32) + fcb_ref[...], 0.0)
    dis = _dis_block(degp_ref[...])
    g1_ref[...] = jnp.dot(xb, c1W_ref[...], preferred_element_type=jnp.float32) * dis


def _t2_body(x1_ref, acc1_ref, degp_ref, c1b_ref, fc1Wa_ref, fc1Wb_ref,
             fc1b_ref, c2Wa_ref, c2Wb_ref, x5_ref, g2_ref):
    dis = _dis_block(degp_ref[...])
    s1 = acc1_ref[0] + acc1_ref[1]
    x2 = jnp.maximum(dis * s1 + c1b_ref[...], 0.0)
    x1b = x1_ref[...]
    x5_ref[...] = jnp.maximum(
        jnp.dot(x1b, fc1Wa_ref[...], preferred_element_type=jnp.float32)
        + jnp.dot(x2, fc1Wb_ref[...], preferred_element_type=jnp.float32)
        + fc1b_ref[...], 0.0)
    g2_ref[...] = (jnp.dot(x1b, c2Wa_ref[...], preferred_element_type=jnp.float32)
                   + jnp.dot(x2, c2Wb_ref[...], preferred_element_type=jnp.float32)) * dis


def _t3_body(x5_ref, acc2_ref, degp_ref, c2b_ref, W9_ref, b9_ref, W3_ref,
             x9_ref, g3_ref):
    dis = _dis_block(degp_ref[...])
    s2 = acc2_ref[0] + acc2_ref[1]
    x6 = jnp.maximum(dis * s2 + c2b_ref[...], 0.0)
    x7 = x5_ref[...] + x6
    x9_ref[...] = jnp.dot(x7, W9_ref[...], preferred_element_type=jnp.float32) + b9_ref[...]
    g3_ref[...] = jnp.dot(x7, W3_ref[...], preferred_element_type=jnp.float32) * dis


def _t4_body(x9_ref, acc3_ref, degp_ref, c3b_ref, out_ref):
    dis = _dis_block(degp_ref[...])
    s3 = acc3_ref[0] + acc3_ref[1]
    out_ref[...] = x9_ref[...] + dis * s3 + c3b_ref[...]


def _row_spec(w):
    return pl.BlockSpec((_TB, w), lambda i: (i, 0))


def _part_spec(w):
    return pl.BlockSpec((NC, _TB, w), lambda i: (0, i, 0))


def _full_spec(shape):
    nd = len(shape)
    return pl.BlockSpec(shape, lambda i: (0,) * nd)


def _sds(shape):
    return jax.ShapeDtypeStruct(shape, jnp.float32)


_t1_call = pl.pallas_call(
    _t1_body,
    grid=_GRID,
    in_specs=[_row_spec(128), _full_spec((128, 128)), _full_spec((1, 128)),
              _full_spec((128, 128)), _part_spec(PAD_W)],
    out_specs=[_row_spec(128), _row_spec(128)],
    out_shape=[_sds((N, 128)), _sds((N, 128))],
)

_t2_call = pl.pallas_call(
    _t2_body,
    grid=_GRID,
    in_specs=[_row_spec(128), _part_spec(128), _part_spec(PAD_W),
              _full_spec((1, 128)), _full_spec((128, 64)), _full_spec((128, 64)),
              _full_spec((1, 64)), _full_spec((128, 64)), _full_spec((128, 64))],
    out_specs=[_row_spec(64), _row_spec(64)],
    out_shape=[_sds((N, 64)), _sds((N, 64))],
)

_t3_call = pl.pallas_call(
    _t3_body,
    grid=_GRID,
    in_specs=[_row_spec(64), _part_spec(64), _part_spec(PAD_W),
              _full_spec((1, 64)), _full_spec((64, PAD_W)),
              _full_spec((1, PAD_W)), _full_spec((64, PAD_W))],
    out_specs=[_row_spec(PAD_W), _row_spec(PAD_W)],
    out_shape=[_sds((N, PAD_W)), _sds((N, PAD_W))],
)

_t4_call = pl.pallas_call(
    _t4_body,
    grid=_GRID,
    in_specs=[_row_spec(PAD_W), _part_spec(PAD_W), _part_spec(PAD_W),
              _full_spec((1, PAD_W))],
    out_specs=_row_spec(PAD_W),
    out_shape=_sds((N, PAD_W)),
)


def _pad16(a):
    return jnp.pad(a, ((0, 0), (0, PAD_W - a.shape[1])))


def kernel(x, edge_index, fc_W, fc_b, conv1_W, conv1_b, fc1_W, fc1_b,
           conv2_W, conv2_b, fc2_W, fc2_b, conv3_W, conv3_b):
    npad = E_PAD - E
    pad_dst = N + (jnp.arange(npad, dtype=jnp.int32) % (N_ACC - N))
    src_p = jnp.concatenate([edge_index[0], jnp.zeros((npad,), jnp.int32)])
    dst_p = jnp.concatenate([edge_index[1], pad_dst])
    src_r = src_p.reshape(NC, NS, NCHUNK, K)
    dst_r = dst_p.reshape(NC, NS, NCHUNK, K)

    ones = jnp.ones((K, PAD_W), jnp.float32)
    zeros16 = jnp.zeros((N_ACC, PAD_W), jnp.float32)
    zeros64 = jnp.zeros((N_ACC, 64), jnp.float32)
    zeros128 = jnp.zeros((N_ACC, 128), jnp.float32)

    degp = _deg_call(dst_r, ones, zeros16)

    x1, g1 = _t1_call(x, fc_W, fc_b.reshape(1, -1), conv1_W, degp)
    acc1 = _spmm128(g1, src_r, dst_r, zeros128)

    x5, g2 = _t2_call(x1, acc1, degp, conv1_b.reshape(1, -1),
                      fc1_W[:128], fc1_W[128:], fc1_b.reshape(1, -1),
                      conv2_W[:128], conv2_W[128:])
    acc2 = _spmm64(g2, src_r, dst_r, zeros64)

    x9p, g3 = _t3_call(x5, acc2, degp, conv2_b.reshape(1, -1),
                       _pad16(fc2_W), _pad16(fc2_b.reshape(1, 1)), _pad16(conv3_W))
    acc3 = _spmm16(g3, src_r, dst_r, zeros16)

    outp = _t4_call(x9p, acc3, degp, _pad16(conv3_b.reshape(1, 1)))
    return outp[:, 0:1]


# speedup vs baseline: 1.7923x; 1.6926x over previous
import functools

import jax
import jax.numpy as jnp
from jax import lax
from jax.experimental import pallas as pl
from jax.experimental.pallas import tpu as pltpu
from jax.experimental.pallas import tpu_sc as plsc

N = 10000
E = 320000
NC = 2
NS = 16
NW = NC * NS
K = 80
NCHUNK = 125
E_PT = NCHUNK * K
N_ACC = 10240
ROWS_PT = N_ACC // NS
PAD_W = 16

_MESH = dict(core_axis_name="c", subcore_axis_name="s")


def _make_degree_kernel():
    @functools.partial(
        pl.kernel,
        mesh=plsc.VectorSubcoreMesh(**_MESH),
        compiler_params=pltpu.CompilerParams(use_tc_tiling_on_sc=False),
        out_type=jax.ShapeDtypeStruct((NC, N_ACC, PAD_W), jnp.float32),
        scratch_types=[
            pltpu.VMEM((NCHUNK, K), jnp.int32),
            pltpu.VMEM((K, PAD_W), jnp.float32),
            pltpu.VMEM_SHARED((N_ACC, PAD_W), jnp.float32),
        ],
    )
    def deg_kernel(dst_hbm, ones_hbm, zeros_hbm, out_hbm, dst_v, ones_v, acc_sh):
        cid = lax.axis_index("c")
        sid = lax.axis_index("s")
        pltpu.sync_copy(dst_hbm.at[cid, sid], dst_v)
        pltpu.sync_copy(ones_hbm, ones_v)
        sl = pl.ds(sid * ROWS_PT, ROWS_PT)
        pltpu.sync_copy(zeros_hbm.at[sl], acc_sh.at[sl])
        plsc.subcore_barrier()

        def body(j, carry):
            pltpu.sync_copy(ones_v, acc_sh.at[dst_v.at[j]], add=True)
            return carry

        lax.fori_loop(0, NCHUNK, body, 0)
        plsc.subcore_barrier()
        pltpu.sync_copy(acc_sh.at[sl], out_hbm.at[cid, sl])

    return deg_kernel


def _make_spmm_kernel(D):
    @functools.partial(
        pl.kernel,
        mesh=plsc.VectorSubcoreMesh(**_MESH),
        compiler_params=pltpu.CompilerParams(use_tc_tiling_on_sc=False),
        out_type=jax.ShapeDtypeStruct((NC, N_ACC, D), jnp.float32),
        scratch_types=[
            pltpu.VMEM((NCHUNK, K), jnp.int32),
            pltpu.VMEM((NCHUNK, K), jnp.int32),
            pltpu.VMEM((K, D), jnp.float32),
            pltpu.VMEM((K, D), jnp.float32),
            pltpu.VMEM_SHARED((N_ACC, D), jnp.float32),
            pltpu.SemaphoreType.DMA,
            pltpu.SemaphoreType.DMA,
        ],
    )
    def spmm(g_hbm, src_hbm, dst_hbm, zeros_hbm, out_hbm,
             src_v, dst_v, rows0, rows1, acc_sh, gs0, gs1):
        cid = lax.axis_index("c")
        sid = lax.axis_index("s")
        pltpu.sync_copy(src_hbm.at[cid, sid], src_v)
        pltpu.sync_copy(dst_hbm.at[cid, sid], dst_v)
        sl = pl.ds(sid * ROWS_PT, ROWS_PT)
        pltpu.sync_copy(zeros_hbm.at[sl], acc_sh.at[sl])
        plsc.subcore_barrier()

        pltpu.async_copy(g_hbm.at[src_v.at[0]], rows0, gs0)
        npair = NCHUNK // 2

        def body(i, carry):
            j0 = 2 * i
            j1 = 2 * i + 1
            pltpu.make_async_copy(g_hbm.at[src_v.at[j0]], rows0, gs0).wait()
            pltpu.async_copy(g_hbm.at[src_v.at[j1]], rows1, gs1)
            pltpu.sync_copy(rows0, acc_sh.at[dst_v.at[j0]], add=True)
            pltpu.make_async_copy(g_hbm.at[src_v.at[j1]], rows1, gs1).wait()

            @pl.when(j1 + 1 < NCHUNK)
            def _():
                pltpu.async_copy(g_hbm.at[src_v.at[j1 + 1]], rows0, gs0)

            pltpu.sync_copy(rows1, acc_sh.at[dst_v.at[j1]], add=True)
            return carry

        lax.fori_loop(0, npair, body, 0)
        if NCHUNK % 2:
            j = NCHUNK - 1
            pltpu.make_async_copy(g_hbm.at[src_v.at[j]], rows0, gs0).wait()
            pltpu.sync_copy(rows0, acc_sh.at[dst_v.at[j]], add=True)
        plsc.subcore_barrier()
        pltpu.sync_copy(acc_sh.at[sl], out_hbm.at[cid, sl])

    return spmm


_deg_call = _make_degree_kernel()
_spmm128 = _make_spmm_kernel(128)
_spmm64 = _make_spmm_kernel(64)
_spmm16 = _make_spmm_kernel(PAD_W)


_TB = 1000
_GRID = (N // _TB,)


def _dis_block(degp):
    deg = degp[0, :, 0:1] + degp[1, :, 0:1]
    return jnp.where(deg > 0.0, lax.rsqrt(deg), 0.0)


def _t1_body(x_ref, fcW_ref, fcb_ref, c1W_ref, degp_ref, x1_ref, g1_ref):
    xb = x_ref[...]
    x1_ref[...] = jnp.maximum(
        jnp.dot(xb, fcW_ref[...], preferred_element_type=jnp.float32) + fcb_ref[...], 0.0)
    dis = _dis_block(degp_ref[...])
    g1_ref[...] = jnp.dot(xb, c1W_ref[...], preferred_element_type=jnp.float32) * dis


def _t2_body(x1_ref, acc1_ref, degp_ref, c1b_ref, fc1Wa_ref, fc1Wb_ref,
             fc1b_ref, c2Wa_ref, c2Wb_ref, x5_ref, g2_ref):
    dis = _dis_block(degp_ref[...])
    s1 = acc1_ref[0] + acc1_ref[1]
    x2 = jnp.maximum(dis * s1 + c1b_ref[...], 0.0)
    x1b = x1_ref[...]
    x5_ref[...] = jnp.maximum(
        jnp.dot(x1b, fc1Wa_ref[...], preferred_element_type=jnp.float32)
        + jnp.dot(x2, fc1Wb_ref[...], preferred_element_type=jnp.float32)
        + fc1b_ref[...], 0.0)
    g2_ref[...] = (jnp.dot(x1b, c2Wa_ref[...], preferred_element_type=jnp.float32)
                   + jnp.dot(x2, c2Wb_ref[...], preferred_element_type=jnp.float32)) * dis


def _t3_body(x5_ref, acc2_ref, degp_ref, c2b_ref, W9_ref, b9_ref, W3_ref,
             x9_ref, g3_ref):
    dis = _dis_block(degp_ref[...])
    s2 = acc2_ref[0] + acc2_ref[1]
    x6 = jnp.maximum(dis * s2 + c2b_ref[...], 0.0)
    x7 = x5_ref[...] + x6
    x9_ref[...] = jnp.dot(x7, W9_ref[...], preferred_element_type=jnp.float32) + b9_ref[...]
    g3_ref[...] = jnp.dot(x7, W3_ref[...], preferred_element_type=jnp.float32) * dis


def _t4_body(x9_ref, acc3_ref, degp_ref, c3b_ref, out_ref):
    dis = _dis_block(degp_ref[...])
    s3 = acc3_ref[0] + acc3_ref[1]
    out_ref[...] = x9_ref[...] + dis * s3 + c3b_ref[...]


def _row_spec(w):
    return pl.BlockSpec((_TB, w), lambda i: (i, 0))


def _part_spec(w):
    return pl.BlockSpec((NC, _TB, w), lambda i: (0, i, 0))


def _full_spec(shape):
    nd = len(shape)
    return pl.BlockSpec(shape, lambda i: (0,) * nd)


def _sds(shape):
    return jax.ShapeDtypeStruct(shape, jnp.float32)


_t1_call = pl.pallas_call(
    _t1_body,
    grid=_GRID,
    in_specs=[_row_spec(128), _full_spec((128, 128)), _full_spec((1, 128)),
              _full_spec((128, 128)), _part_spec(PAD_W)],
    out_specs=[_row_spec(128), _row_spec(128)],
    out_shape=[_sds((N, 128)), _sds((N, 128))],
)

_t2_call = pl.pallas_call(
    _t2_body,
    grid=_GRID,
    in_specs=[_row_spec(128), _part_spec(128), _part_spec(PAD_W),
              _full_spec((1, 128)), _full_spec((128, 64)), _full_spec((128, 64)),
              _full_spec((1, 64)), _full_spec((128, 64)), _full_spec((128, 64))],
    out_specs=[_row_spec(64), _row_spec(64)],
    out_shape=[_sds((N, 64)), _sds((N, 64))],
)

_t3_call = pl.pallas_call(
    _t3_body,
    grid=_GRID,
    in_specs=[_row_spec(64), _part_spec(64), _part_spec(PAD_W),
              _full_spec((1, 64)), _full_spec((64, PAD_W)),
              _full_spec((1, PAD_W)), _full_spec((64, PAD_W))],
    out_specs=[_row_spec(PAD_W), _row_spec(PAD_W)],
    out_shape=[_sds((N, PAD_W)), _sds((N, PAD_W))],
)

_t4_call = pl.pallas_call(
    _t4_body,
    grid=_GRID,
    in_specs=[_row_spec(PAD_W), _part_spec(PAD_W), _part_spec(PAD_W),
              _full_spec((1, PAD_W))],
    out_specs=_row_spec(PAD_W),
    out_shape=_sds((N, PAD_W)),
)


def _pad16(a):
    return jnp.pad(a, ((0, 0), (0, PAD_W - a.shape[1])))


def kernel(x, edge_index, fc_W, fc_b, conv1_W, conv1_b, fc1_W, fc1_b,
           conv2_W, conv2_b, fc2_W, fc2_b, conv3_W, conv3_b):
    src_r = edge_index[0].reshape(NC, NS, NCHUNK, K)
    dst_r = edge_index[1].reshape(NC, NS, NCHUNK, K)

    ones = jnp.ones((K, PAD_W), jnp.float32)
    zeros16 = jnp.zeros((N_ACC, PAD_W), jnp.float32)
    zeros64 = jnp.zeros((N_ACC, 64), jnp.float32)
    zeros128 = jnp.zeros((N_ACC, 128), jnp.float32)

    degp = _deg_call(dst_r, ones, zeros16)

    x1, g1 = _t1_call(x, fc_W, fc_b.reshape(1, -1), conv1_W, degp)
    acc1 = _spmm128(g1, src_r, dst_r, zeros128)

    x5, g2 = _t2_call(x1, acc1, degp, conv1_b.reshape(1, -1),
                      fc1_W[:128], fc1_W[128:], fc1_b.reshape(1, -1),
                      conv2_W[:128], conv2_W[128:])
    acc2 = _spmm64(g2, src_r, dst_r, zeros64)

    x9p, g3 = _t3_call(x5, acc2, degp, conv2_b.reshape(1, -1),
                       _pad16(fc2_W), _pad16(fc2_b.reshape(1, 1)), _pad16(conv3_W))
    acc3 = _spmm16(g3, src_r, dst_r, zeros16)

    outp = _t4_call(x9p, acc3, degp, _pad16(conv3_b.reshape(1, 1)))
    return outp[:, 0:1]


# speedup vs baseline: 2.0360x; 1.1360x over previous
import functools

import jax
import jax.numpy as jnp
from jax import lax
from jax.experimental import pallas as pl
from jax.experimental.pallas import tpu as pltpu
from jax.experimental.pallas import tpu_sc as plsc

N = 10000
E = 320000
NC = 2
NS = 16
NW = NC * NS
K = 80
NCHUNK = 125
E_PT = NCHUNK * K
N_ACC = 10240
ROWS_PT = N_ACC // NS
PAD_W = 16

_MESH = dict(core_axis_name="c", subcore_axis_name="s")


def _make_degree_kernel():
    @functools.partial(
        pl.kernel,
        mesh=plsc.VectorSubcoreMesh(**_MESH),
        compiler_params=pltpu.CompilerParams(use_tc_tiling_on_sc=False),
        out_type=jax.ShapeDtypeStruct((NC, N_ACC, PAD_W), jnp.float32),
        scratch_types=[
            pltpu.VMEM((NCHUNK, K), jnp.int32),
            pltpu.VMEM((K, PAD_W), jnp.float32),
            pltpu.VMEM_SHARED((N_ACC, PAD_W), jnp.float32),
        ],
    )
    def deg_kernel(dst_hbm, ones_hbm, zeros_hbm, out_hbm, dst_v, ones_v, acc_sh):
        cid = lax.axis_index("c")
        sid = lax.axis_index("s")
        pltpu.sync_copy(dst_hbm.at[cid, sid], dst_v)
        pltpu.sync_copy(ones_hbm, ones_v)
        sl = pl.ds(sid * ROWS_PT, ROWS_PT)
        pltpu.sync_copy(zeros_hbm.at[sl], acc_sh.at[sl])
        plsc.subcore_barrier()

        def body(j, carry):
            pltpu.sync_copy(ones_v, acc_sh.at[dst_v.at[j]], add=True)
            return carry

        lax.fori_loop(0, NCHUNK, body, 0)
        plsc.subcore_barrier()
        pltpu.sync_copy(acc_sh.at[sl], out_hbm.at[cid, sl])

    return deg_kernel


def _make_spmm_kernel(D):
    @functools.partial(
        pl.kernel,
        mesh=plsc.VectorSubcoreMesh(**_MESH),
        compiler_params=pltpu.CompilerParams(use_tc_tiling_on_sc=False),
        out_type=jax.ShapeDtypeStruct((NC, N_ACC, D), jnp.float32),
        scratch_types=[
            pltpu.VMEM((NCHUNK, K), jnp.int32),
            pltpu.VMEM((NCHUNK, K), jnp.int32),
            pltpu.VMEM((K, D), jnp.float32),
            pltpu.VMEM((K, D), jnp.float32),
            pltpu.VMEM_SHARED((N_ACC, D), jnp.float32),
            pltpu.SemaphoreType.DMA,
            pltpu.SemaphoreType.DMA,
        ],
    )
    def spmm(g_hbm, src_hbm, dst_hbm, zeros_hbm, out_hbm,
             src_v, dst_v, rows0, rows1, acc_sh, gs0, gs1):
        cid = lax.axis_index("c")
        sid = lax.axis_index("s")
        pltpu.sync_copy(src_hbm.at[cid, sid], src_v)
        pltpu.sync_copy(dst_hbm.at[cid, sid], dst_v)
        sl = pl.ds(sid * ROWS_PT, ROWS_PT)
        pltpu.sync_copy(zeros_hbm.at[sl], acc_sh.at[sl])
        plsc.subcore_barrier()

        pltpu.async_copy(g_hbm.at[src_v.at[0]], rows0, gs0)
        npair = NCHUNK // 2

        def body(i, carry):
            j0 = 2 * i
            j1 = 2 * i + 1
            pltpu.make_async_copy(g_hbm.at[src_v.at[j0]], rows0, gs0).wait()
            pltpu.async_copy(g_hbm.at[src_v.at[j1]], rows1, gs1)
            pltpu.sync_copy(rows0, acc_sh.at[dst_v.at[j0]], add=True)
            pltpu.make_async_copy(g_hbm.at[src_v.at[j1]], rows1, gs1).wait()

            @pl.when(j1 + 1 < NCHUNK)
            def _():
                pltpu.async_copy(g_hbm.at[src_v.at[j1 + 1]], rows0, gs0)

            pltpu.sync_copy(rows1, acc_sh.at[dst_v.at[j1]], add=True)
            return carry

        lax.fori_loop(0, npair, body, 0)
        if NCHUNK % 2:
            j = NCHUNK - 1
            pltpu.make_async_copy(g_hbm.at[src_v.at[j]], rows0, gs0).wait()
            pltpu.sync_copy(rows0, acc_sh.at[dst_v.at[j]], add=True)
        plsc.subcore_barrier()
        pltpu.sync_copy(acc_sh.at[sl], out_hbm.at[cid, sl])

    return spmm


def _make_spmm1_kernel():
    @functools.partial(
        pl.kernel,
        mesh=plsc.VectorSubcoreMesh(**_MESH),
        compiler_params=pltpu.CompilerParams(use_tc_tiling_on_sc=False,
                                             needs_layout_passes=False),
        out_type=jax.ShapeDtypeStruct((NC, N_ACC, PAD_W), jnp.float32),
        scratch_types=[
            pltpu.VMEM((E_PT,), jnp.int32),
            pltpu.VMEM((NCHUNK, K), jnp.int32),
            pltpu.VMEM((N,), jnp.float32),
            pltpu.VMEM((K, PAD_W), jnp.float32),
            pltpu.VMEM_SHARED((N_ACC, PAD_W), jnp.float32),
        ],
    )
    def spmm1(g_hbm, src_hbm, dst_hbm, zeros_hbm, out_hbm,
              src_v, dst_v, g_v, rows_v, acc_sh):
        cid = lax.axis_index("c")
        sid = lax.axis_index("s")
        pltpu.sync_copy(src_hbm.at[cid, sid], src_v)
        pltpu.sync_copy(dst_hbm.at[cid, sid], dst_v)
        pltpu.sync_copy(g_hbm, g_v)
        pltpu.sync_copy(zeros_hbm.at[pl.ds(0, K)], rows_v)
        sl = pl.ds(sid * ROWS_PT, ROWS_PT)
        pltpu.sync_copy(zeros_hbm.at[sl], acc_sh.at[sl])
        plsc.subcore_barrier()

        iota = lax.iota(jnp.int32, 16)
        zcol = jnp.zeros((16,), jnp.int32)

        def body(j, carry):
            for g in range(K // 16):
                s = src_v[pl.ds(j * K + g * 16, 16)]
                v = plsc.load_gather(g_v, [s])
                plsc.store_scatter(rows_v, [iota + (g * 16), zcol], v)
            pltpu.sync_copy(rows_v, acc_sh.at[dst_v.at[j]], add=True)
            return carry

        lax.fori_loop(0, NCHUNK, body, 0)
        plsc.subcore_barrier()
        pltpu.sync_copy(acc_sh.at[sl], out_hbm.at[cid, sl])

    return spmm1


_deg_call = _make_degree_kernel()
_spmm128 = _make_spmm_kernel(128)
_spmm64 = _make_spmm_kernel(64)
_spmm16 = _make_spmm1_kernel()


_TB = 1000
_GRID = (N // _TB,)


def _dis_block(degp):
    deg = degp[0, :, 0:1] + degp[1, :, 0:1]
    return jnp.where(deg > 0.0, lax.rsqrt(deg), 0.0)


def _t1_body(x_ref, fcW_ref, fcb_ref, c1W_ref, degp_ref, x1_ref, g1_ref):
    xb = x_ref[...]
    x1_ref[...] = jnp.maximum(
        jnp.dot(xb, fcW_ref[...], preferred_element_type=jnp.float32) + fcb_ref[...], 0.0)
    dis = _dis_block(degp_ref[...])
    g1_ref[...] = jnp.dot(xb, c1W_ref[...], preferred_element_type=jnp.float32) * dis


def _t2_body(x1_ref, acc1_ref, degp_ref, c1b_ref, fc1Wa_ref, fc1Wb_ref,
             fc1b_ref, c2Wa_ref, c2Wb_ref, x5_ref, g2_ref):
    dis = _dis_block(degp_ref[...])
    s1 = acc1_ref[0] + acc1_ref[1]
    x2 = jnp.maximum(dis * s1 + c1b_ref[...], 0.0)
    x1b = x1_ref[...]
    x5_ref[...] = jnp.maximum(
        jnp.dot(x1b, fc1Wa_ref[...], preferred_element_type=jnp.float32)
        + jnp.dot(x2, fc1Wb_ref[...], preferred_element_type=jnp.float32)
        + fc1b_ref[...], 0.0)
    g2_ref[...] = (jnp.dot(x1b, c2Wa_ref[...], preferred_element_type=jnp.float32)
                   + jnp.dot(x2, c2Wb_ref[...], preferred_element_type=jnp.float32)) * dis


def _t3_body(x5_ref, acc2_ref, degp_ref, c2b_ref, W9_ref, b9_ref, W3_ref,
             x9_ref, g3_ref):
    dis = _dis_block(degp_ref[...])
    s2 = acc2_ref[0] + acc2_ref[1]
    x6 = jnp.maximum(dis * s2 + c2b_ref[...], 0.0)
    x7 = x5_ref[...] + x6
    x9_ref[...] = jnp.dot(x7, W9_ref[...], preferred_element_type=jnp.float32) + b9_ref[...]
    g3_ref[...] = jnp.dot(x7, W3_ref[...], preferred_element_type=jnp.float32) * dis


def _t4_body(x9_ref, acc3_ref, degp_ref, c3b_ref, out_ref):
    dis = _dis_block(degp_ref[...])
    s3 = acc3_ref[0] + acc3_ref[1]
    out_ref[...] = x9_ref[...] + dis * s3 + c3b_ref[...]


def _row_spec(w):
    return pl.BlockSpec((_TB, w), lambda i: (i, 0))


def _part_spec(w):
    return pl.BlockSpec((NC, _TB, w), lambda i: (0, i, 0))


def _full_spec(shape):
    nd = len(shape)
    return pl.BlockSpec(shape, lambda i: (0,) * nd)


def _sds(shape):
    return jax.ShapeDtypeStruct(shape, jnp.float32)


_t1_call = pl.pallas_call(
    _t1_body,
    grid=_GRID,
    in_specs=[_row_spec(128), _full_spec((128, 128)), _full_spec((1, 128)),
              _full_spec((128, 128)), _part_spec(PAD_W)],
    out_specs=[_row_spec(128), _row_spec(128)],
    out_shape=[_sds((N, 128)), _sds((N, 128))],
)

_t2_call = pl.pallas_call(
    _t2_body,
    grid=_GRID,
    in_specs=[_row_spec(128), _part_spec(128), _part_spec(PAD_W),
              _full_spec((1, 128)), _full_spec((128, 64)), _full_spec((128, 64)),
              _full_spec((1, 64)), _full_spec((128, 64)), _full_spec((128, 64))],
    out_specs=[_row_spec(64), _row_spec(64)],
    out_shape=[_sds((N, 64)), _sds((N, 64))],
)

_t3_call = pl.pallas_call(
    _t3_body,
    grid=_GRID,
    in_specs=[_row_spec(64), _part_spec(64), _part_spec(PAD_W),
              _full_spec((1, 64)), _full_spec((64, PAD_W)),
              _full_spec((1, PAD_W)), _full_spec((64, 1))],
    out_specs=[_row_spec(PAD_W), _row_spec(1)],
    out_shape=[_sds((N, PAD_W)), _sds((N, 1))],
)

_t4_call = pl.pallas_call(
    _t4_body,
    grid=_GRID,
    in_specs=[_row_spec(PAD_W), _part_spec(PAD_W), _part_spec(PAD_W),
              _full_spec((1, PAD_W))],
    out_specs=_row_spec(PAD_W),
    out_shape=_sds((N, PAD_W)),
)


def _pad16(a):
    return jnp.pad(a, ((0, 0), (0, PAD_W - a.shape[1])))


def kernel(x, edge_index, fc_W, fc_b, conv1_W, conv1_b, fc1_W, fc1_b,
           conv2_W, conv2_b, fc2_W, fc2_b, conv3_W, conv3_b):
    src_r = edge_index[0].reshape(NC, NS, NCHUNK, K)
    dst_r = edge_index[1].reshape(NC, NS, NCHUNK, K)
    src_f = edge_index[0].reshape(NC, NS, E_PT)

    ones = jnp.ones((K, PAD_W), jnp.float32)
    zeros16 = jnp.zeros((N_ACC, PAD_W), jnp.float32)
    zeros64 = jnp.zeros((N_ACC, 64), jnp.float32)
    zeros128 = jnp.zeros((N_ACC, 128), jnp.float32)

    degp = _deg_call(dst_r, ones, zeros16)

    x1, g1 = _t1_call(x, fc_W, fc_b.reshape(1, -1), conv1_W, degp)
    acc1 = _spmm128(g1, src_r, dst_r, zeros128)

    x5, g2 = _t2_call(x1, acc1, degp, conv1_b.reshape(1, -1),
                      fc1_W[:128], fc1_W[128:], fc1_b.reshape(1, -1),
                      conv2_W[:128], conv2_W[128:])
    acc2 = _spmm64(g2, src_r, dst_r, zeros64)

    x9p, g3 = _t3_call(x5, acc2, degp, conv2_b.reshape(1, -1),
                       _pad16(fc2_W), _pad16(fc2_b.reshape(1, 1)), conv3_W)
    acc3 = _spmm16(g3.reshape(-1), src_f, dst_r, zeros16)

    outp = _t4_call(x9p, acc3, degp, _pad16(conv3_b.reshape(1, 1)))
    return outp[:, 0:1]


# speedup vs baseline: 2.2033x; 1.0822x over previous
import functools

import jax
import jax.numpy as jnp
from jax import lax
from jax.experimental import pallas as pl
from jax.experimental.pallas import tpu as pltpu
from jax.experimental.pallas import tpu_sc as plsc

N = 10000
E = 320000
NC = 2
NS = 16
NW = NC * NS
K = 80
NCHUNK = 125
E_PT = NCHUNK * K
N_ACC = 10240
ROWS_PT = N_ACC // NS
PAD_W = 16

_MESH = dict(core_axis_name="c", subcore_axis_name="s")


def _make_degree_kernel():
    @functools.partial(
        pl.kernel,
        mesh=plsc.VectorSubcoreMesh(**_MESH),
        compiler_params=pltpu.CompilerParams(use_tc_tiling_on_sc=False),
        out_type=jax.ShapeDtypeStruct((NC, N_ACC, PAD_W), jnp.float32),
        scratch_types=[
            pltpu.VMEM((NCHUNK, K), jnp.int32),
            pltpu.VMEM((K, PAD_W), jnp.float32),
            pltpu.VMEM_SHARED((N_ACC, PAD_W), jnp.float32),
        ],
    )
    def deg_kernel(dst_hbm, ones_hbm, zeros_hbm, out_hbm, dst_v, ones_v, acc_sh):
        cid = lax.axis_index("c")
        sid = lax.axis_index("s")
        pltpu.sync_copy(dst_hbm.at[cid, sid], dst_v)
        pltpu.sync_copy(ones_hbm, ones_v)
        sl = pl.ds(sid * ROWS_PT, ROWS_PT)
        pltpu.sync_copy(zeros_hbm.at[sl], acc_sh.at[sl])
        plsc.subcore_barrier()

        def body(j, carry):
            pltpu.sync_copy(ones_v, acc_sh.at[dst_v.at[j]], add=True)
            return carry

        lax.fori_loop(0, NCHUNK, body, 0)
        plsc.subcore_barrier()
        pltpu.sync_copy(acc_sh.at[sl], out_hbm.at[cid, sl])

    return deg_kernel


def _make_spmm_kernel(D):
    @functools.partial(
        pl.kernel,
        mesh=plsc.VectorSubcoreMesh(**_MESH),
        compiler_params=pltpu.CompilerParams(use_tc_tiling_on_sc=False),
        out_type=jax.ShapeDtypeStruct((NC, N_ACC, D), jnp.float32),
        scratch_types=[
            pltpu.VMEM((NCHUNK, K), jnp.int32),
            pltpu.VMEM((NCHUNK, K), jnp.int32),
            pltpu.VMEM((K, D), jnp.float32),
            pltpu.VMEM((K, D), jnp.float32),
            pltpu.VMEM_SHARED((N_ACC, D), jnp.float32),
            pltpu.SemaphoreType.DMA,
            pltpu.SemaphoreType.DMA,
        ],
    )
    def spmm(g_hbm, src_hbm, dst_hbm, zeros_hbm, out_hbm,
             src_v, dst_v, rows0, rows1, acc_sh, gs0, gs1):
        cid = lax.axis_index("c")
        sid = lax.axis_index("s")
        pltpu.sync_copy(src_hbm.at[cid, sid], src_v)
        pltpu.sync_copy(dst_hbm.at[cid, sid], dst_v)
        sl = pl.ds(sid * ROWS_PT, ROWS_PT)
        pltpu.sync_copy(zeros_hbm.at[sl], acc_sh.at[sl])
        plsc.subcore_barrier()

        pltpu.async_copy(g_hbm.at[src_v.at[0]], rows0, gs0)
        npair = NCHUNK // 2

        def body(i, carry):
            j0 = 2 * i
            j1 = 2 * i + 1
            pltpu.make_async_copy(g_hbm.at[src_v.at[j0]], rows0, gs0).wait()
            pltpu.async_copy(g_hbm.at[src_v.at[j1]], rows1, gs1)
            pltpu.sync_copy(rows0, acc_sh.at[dst_v.at[j0]], add=True)
            pltpu.make_async_copy(g_hbm.at[src_v.at[j1]], rows1, gs1).wait()

            @pl.when(j1 + 1 < NCHUNK)
            def _():
                pltpu.async_copy(g_hbm.at[src_v.at[j1 + 1]], rows0, gs0)

            pltpu.sync_copy(rows1, acc_sh.at[dst_v.at[j1]], add=True)
            return carry

        lax.fori_loop(0, npair, body, 0)
        if NCHUNK % 2:
            j = NCHUNK - 1
            pltpu.make_async_copy(g_hbm.at[src_v.at[j]], rows0, gs0).wait()
            pltpu.sync_copy(rows0, acc_sh.at[dst_v.at[j]], add=True)
        plsc.subcore_barrier()
        pltpu.sync_copy(acc_sh.at[sl], out_hbm.at[cid, sl])

    return spmm


def _make_spmm64_spmem_kernel():
    D = 64

    @functools.partial(
        pl.kernel,
        mesh=plsc.VectorSubcoreMesh(**_MESH),
        compiler_params=pltpu.CompilerParams(use_tc_tiling_on_sc=False),
        out_type=jax.ShapeDtypeStruct((NC, N_ACC, D), jnp.float32),
        scratch_types=[
            pltpu.VMEM((NCHUNK, K), jnp.int32),
            pltpu.VMEM((NCHUNK, K), jnp.int32),
            pltpu.VMEM((K, D), jnp.float32),
            pltpu.VMEM((K, D), jnp.float32),
            pltpu.VMEM_SHARED((N, D), jnp.float32),
            pltpu.VMEM_SHARED((N_ACC, D), jnp.float32),
            pltpu.SemaphoreType.DMA,
            pltpu.SemaphoreType.DMA,
        ],
    )
    def spmm(g_hbm, src_hbm, dst_hbm, zeros_hbm, out_hbm,
             src_v, dst_v, rows0, rows1, g_sh, acc_sh, gs0, gs1):
        cid = lax.axis_index("c")
        sid = lax.axis_index("s")
        pltpu.sync_copy(src_hbm.at[cid, sid], src_v)
        pltpu.sync_copy(dst_hbm.at[cid, sid], dst_v)
        sl = pl.ds(sid * ROWS_PT, ROWS_PT)
        pltpu.sync_copy(zeros_hbm.at[sl], acc_sh.at[sl])

        @pl.when(sid < 10)
        def _():
            gl = pl.ds(sid * 1000, 1000)
            pltpu.sync_copy(g_hbm.at[gl], g_sh.at[gl])

        plsc.subcore_barrier()

        pltpu.async_copy(g_sh.at[src_v.at[0]], rows0, gs0)
        npair = NCHUNK // 2

        def body(i, carry):
            j0 = 2 * i
            j1 = 2 * i + 1
            pltpu.make_async_copy(g_sh.at[src_v.at[j0]], rows0, gs0).wait()
            pltpu.async_copy(g_sh.at[src_v.at[j1]], rows1, gs1)
            pltpu.sync_copy(rows0, acc_sh.at[dst_v.at[j0]], add=True)
            pltpu.make_async_copy(g_sh.at[src_v.at[j1]], rows1, gs1).wait()

            @pl.when(j1 + 1 < NCHUNK)
            def _():
                pltpu.async_copy(g_sh.at[src_v.at[j1 + 1]], rows0, gs0)

            pltpu.sync_copy(rows1, acc_sh.at[dst_v.at[j1]], add=True)
            return carry

        lax.fori_loop(0, npair, body, 0)
        if NCHUNK % 2:
            j = NCHUNK - 1
            pltpu.make_async_copy(g_sh.at[src_v.at[j]], rows0, gs0).wait()
            pltpu.sync_copy(rows0, acc_sh.at[dst_v.at[j]], add=True)
        plsc.subcore_barrier()
        pltpu.sync_copy(acc_sh.at[sl], out_hbm.at[cid, sl])

    return spmm


def _make_spmm1_kernel():
    @functools.partial(
        pl.kernel,
        mesh=plsc.VectorSubcoreMesh(**_MESH),
        compiler_params=pltpu.CompilerParams(use_tc_tiling_on_sc=False,
                                             needs_layout_passes=False),
        out_type=jax.ShapeDtypeStruct((NC, N_ACC, PAD_W), jnp.float32),
        scratch_types=[
            pltpu.VMEM((E_PT,), jnp.int32),
            pltpu.VMEM((NCHUNK, K), jnp.int32),
            pltpu.VMEM((N,), jnp.float32),
            pltpu.VMEM((K, PAD_W), jnp.float32),
            pltpu.VMEM_SHARED((N_ACC, PAD_W), jnp.float32),
        ],
    )
    def spmm1(g_hbm, src_hbm, dst_hbm, zeros_hbm, out_hbm,
              src_v, dst_v, g_v, rows_v, acc_sh):
        cid = lax.axis_index("c")
        sid = lax.axis_index("s")
        pltpu.sync_copy(src_hbm.at[cid, sid], src_v)
        pltpu.sync_copy(dst_hbm.at[cid, sid], dst_v)
        pltpu.sync_copy(g_hbm, g_v)
        pltpu.sync_copy(zeros_hbm.at[pl.ds(0, K)], rows_v)
        sl = pl.ds(sid * ROWS_PT, ROWS_PT)
        pltpu.sync_copy(zeros_hbm.at[sl], acc_sh.at[sl])
        plsc.subcore_barrier()

        iota = lax.iota(jnp.int32, 16)
        zcol = jnp.zeros((16,), jnp.int32)

        def body(j, carry):
            for g in range(K // 16):
                s = src_v[pl.ds(j * K + g * 16, 16)]
                v = plsc.load_gather(g_v, [s])
                plsc.store_scatter(rows_v, [iota + (g * 16), zcol], v)
            pltpu.sync_copy(rows_v, acc_sh.at[dst_v.at[j]], add=True)
            return carry

        lax.fori_loop(0, NCHUNK, body, 0)
        plsc.subcore_barrier()
        pltpu.sync_copy(acc_sh.at[sl], out_hbm.at[cid, sl])

    return spmm1


_deg_call = _make_degree_kernel()
_spmm128 = _make_spmm_kernel(128)
_spmm64 = _make_spmm64_spmem_kernel()
_spmm16 = _make_spmm1_kernel()


_TB = 1000
_GRID = (N // _TB,)


def _dis_block(degp):
    deg = degp[0, :, 0:1] + degp[1, :, 0:1]
    return jnp.where(deg > 0.0, lax.rsqrt(deg), 0.0)


def _t1_body(x_ref, fcW_ref, fcb_ref, c1W_ref, degp_ref, x1_ref, g1_ref):
    xb = x_ref[...]
    x1_ref[...] = jnp.maximum(
        jnp.dot(xb, fcW_ref[...], preferred_element_type=jnp.float32) + fcb_ref[...], 0.0)
    dis = _dis_block(degp_ref[...])
    g1_ref[...] = jnp.dot(xb, c1W_ref[...], preferred_element_type=jnp.float32) * dis


def _t2_body(x1_ref, acc1_ref, degp_ref, c1b_ref, fc1Wa_ref, fc1Wb_ref,
             fc1b_ref, c2Wa_ref, c2Wb_ref, x5_ref, g2_ref):
    dis = _dis_block(degp_ref[...])
    s1 = acc1_ref[0] + acc1_ref[1]
    x2 = jnp.maximum(dis * s1 + c1b_ref[...], 0.0)
    x1b = x1_ref[...]
    x5_ref[...] = jnp.maximum(
        jnp.dot(x1b, fc1Wa_ref[...], preferred_element_type=jnp.float32)
        + jnp.dot(x2, fc1Wb_ref[...], preferred_element_type=jnp.float32)
        + fc1b_ref[...], 0.0)
    g2_ref[...] = (jnp.dot(x1b, c2Wa_ref[...], preferred_element_type=jnp.float32)
                   + jnp.dot(x2, c2Wb_ref[...], preferred_element_type=jnp.float32)) * dis


def _t3_body(x5_ref, acc2_ref, degp_ref, c2b_ref, W9_ref, b9_ref, W3_ref,
             x9_ref, g3_ref):
    dis = _dis_block(degp_ref[...])
    s2 = acc2_ref[0] + acc2_ref[1]
    x6 = jnp.maximum(dis * s2 + c2b_ref[...], 0.0)
    x7 = x5_ref[...] + x6
    x9_ref[...] = jnp.dot(x7, W9_ref[...], preferred_element_type=jnp.float32) + b9_ref[...]
    g3_ref[...] = jnp.dot(x7, W3_ref[...], preferred_element_type=jnp.float32) * dis


def _t4_body(x9_ref, acc3_ref, degp_ref, c3b_ref, out_ref):
    dis = _dis_block(degp_ref[...])
    s3 = acc3_ref[0] + acc3_ref[1]
    out_ref[...] = x9_ref[...] + dis * s3 + c3b_ref[...]


def _row_spec(w):
    return pl.BlockSpec((_TB, w), lambda i: (i, 0))


def _part_spec(w):
    return pl.BlockSpec((NC, _TB, w), lambda i: (0, i, 0))


def _full_spec(shape):
    nd = len(shape)
    return pl.BlockSpec(shape, lambda i: (0,) * nd)


def _sds(shape):
    return jax.ShapeDtypeStruct(shape, jnp.float32)


_t1_call = pl.pallas_call(
    _t1_body,
    grid=_GRID,
    in_specs=[_row_spec(128), _full_spec((128, 128)), _full_spec((1, 128)),
              _full_spec((128, 128)), _part_spec(PAD_W)],
    out_specs=[_row_spec(128), _row_spec(128)],
    out_shape=[_sds((N, 128)), _sds((N, 128))],
)

_t2_call = pl.pallas_call(
    _t2_body,
    grid=_GRID,
    in_specs=[_row_spec(128), _part_spec(128), _part_spec(PAD_W),
              _full_spec((1, 128)), _full_spec((128, 64)), _full_spec((128, 64)),
              _full_spec((1, 64)), _full_spec((128, 64)), _full_spec((128, 64))],
    out_specs=[_row_spec(64), _row_spec(64)],
    out_shape=[_sds((N, 64)), _sds((N, 64))],
)

_t3_call = pl.pallas_call(
    _t3_body,
    grid=_GRID,
    in_specs=[_row_spec(64), _part_spec(64), _part_spec(PAD_W),
              _full_spec((1, 64)), _full_spec((64, PAD_W)),
              _full_spec((1, PAD_W)), _full_spec((64, 1))],
    out_specs=[_row_spec(PAD_W), _row_spec(1)],
    out_shape=[_sds((N, PAD_W)), _sds((N, 1))],
)

_t4_call = pl.pallas_call(
    _t4_body,
    grid=_GRID,
    in_specs=[_row_spec(PAD_W), _part_spec(PAD_W), _part_spec(PAD_W),
              _full_spec((1, PAD_W))],
    out_specs=_row_spec(PAD_W),
    out_shape=_sds((N, PAD_W)),
)


def _pad16(a):
    return jnp.pad(a, ((0, 0), (0, PAD_W - a.shape[1])))


def kernel(x, edge_index, fc_W, fc_b, conv1_W, conv1_b, fc1_W, fc1_b,
           conv2_W, conv2_b, fc2_W, fc2_b, conv3_W, conv3_b):
    src_r = edge_index[0].reshape(NC, NS, NCHUNK, K)
    dst_r = edge_index[1].reshape(NC, NS, NCHUNK, K)
    src_f = edge_index[0].reshape(NC, NS, E_PT)

    ones = jnp.ones((K, PAD_W), jnp.float32)
    zeros16 = jnp.zeros((N_ACC, PAD_W), jnp.float32)
    zeros64 = jnp.zeros((N_ACC, 64), jnp.float32)
    zeros128 = jnp.zeros((N_ACC, 128), jnp.float32)

    degp = _deg_call(dst_r, ones, zeros16)

    x1, g1 = _t1_call(x, fc_W, fc_b.reshape(1, -1), conv1_W, degp)
    acc1 = _spmm128(g1, src_r, dst_r, zeros128)

    x5, g2 = _t2_call(x1, acc1, degp, conv1_b.reshape(1, -1),
                      fc1_W[:128], fc1_W[128:], fc1_b.reshape(1, -1),
                      conv2_W[:128], conv2_W[128:])
    acc2 = _spmm64(g2, src_r, dst_r, zeros64)

    x9p, g3 = _t3_call(x5, acc2, degp, conv2_b.reshape(1, -1),
                       _pad16(fc2_W), _pad16(fc2_b.reshape(1, 1)), conv3_W)
    acc3 = _spmm16(g3.reshape(-1), src_f, dst_r, zeros16)

    outp = _t4_call(x9p, acc3, degp, _pad16(conv3_b.reshape(1, 1)))
    return outp[:, 0:1]


# speedup vs baseline: 2.2092x; 1.0026x over previous
import functools

import jax
import jax.numpy as jnp
from jax import lax
from jax.experimental import pallas as pl
from jax.experimental.pallas import tpu as pltpu
from jax.experimental.pallas import tpu_sc as plsc

N = 10000
E = 320000
NC = 2
NS = 16
NW = NC * NS
K = 80
NCHUNK = 125
E_PT = NCHUNK * K
N_ACC = 10240
ROWS_PT = N_ACC // NS
PAD_W = 16

_MESH = dict(core_axis_name="c", subcore_axis_name="s")


def _make_degree_kernel():
    @functools.partial(
        pl.kernel,
        mesh=plsc.VectorSubcoreMesh(**_MESH),
        compiler_params=pltpu.CompilerParams(use_tc_tiling_on_sc=False),
        out_type=jax.ShapeDtypeStruct((NC, N_ACC, PAD_W), jnp.float32),
        scratch_types=[
            pltpu.VMEM((NCHUNK, K), jnp.int32),
            pltpu.VMEM((K, PAD_W), jnp.float32),
            pltpu.VMEM_SHARED((N_ACC, PAD_W), jnp.float32),
        ],
    )
    def deg_kernel(dst_hbm, ones_hbm, zeros_hbm, out_hbm, dst_v, ones_v, acc_sh):
        cid = lax.axis_index("c")
        sid = lax.axis_index("s")
        pltpu.sync_copy(dst_hbm.at[cid, sid], dst_v)
        pltpu.sync_copy(ones_hbm, ones_v)
        sl = pl.ds(sid * ROWS_PT, ROWS_PT)
        pltpu.sync_copy(zeros_hbm.at[sl], acc_sh.at[sl])
        plsc.subcore_barrier()

        def body(j, carry):
            pltpu.sync_copy(ones_v, acc_sh.at[dst_v.at[j]], add=True)
            return carry

        lax.fori_loop(0, NCHUNK, body, 0)
        plsc.subcore_barrier()
        pltpu.sync_copy(acc_sh.at[sl], out_hbm.at[cid, sl])

    return deg_kernel


def _make_spmm_kernel(D):
    @functools.partial(
        pl.kernel,
        mesh=plsc.VectorSubcoreMesh(**_MESH),
        compiler_params=pltpu.CompilerParams(use_tc_tiling_on_sc=False),
        out_type=jax.ShapeDtypeStruct((NC, N_ACC, D), jnp.float32),
        scratch_types=[
            pltpu.VMEM((NCHUNK, K), jnp.int32),
            pltpu.VMEM((NCHUNK, K), jnp.int32),
            pltpu.VMEM((K, D), jnp.float32),
            pltpu.VMEM((K, D), jnp.float32),
            pltpu.VMEM_SHARED((N_ACC, D), jnp.float32),
            pltpu.SemaphoreType.DMA,
            pltpu.SemaphoreType.DMA,
        ],
    )
    def spmm(g_hbm, src_hbm, dst_hbm, zeros_hbm, out_hbm,
             src_v, dst_v, rows0, rows1, acc_sh, gs0, gs1):
        cid = lax.axis_index("c")
        sid = lax.axis_index("s")
        pltpu.sync_copy(src_hbm.at[cid, sid], src_v)
        pltpu.sync_copy(dst_hbm.at[cid, sid], dst_v)
        sl = pl.ds(sid * ROWS_PT, ROWS_PT)
        pltpu.sync_copy(zeros_hbm.at[sl], acc_sh.at[sl])
        plsc.subcore_barrier()

        pltpu.async_copy(g_hbm.at[src_v.at[0]], rows0, gs0)
        npair = NCHUNK // 2

        def body(i, carry):
            j0 = 2 * i
            j1 = 2 * i + 1
            pltpu.make_async_copy(g_hbm.at[src_v.at[j0]], rows0, gs0).wait()
            pltpu.async_copy(g_hbm.at[src_v.at[j1]], rows1, gs1)
            pltpu.sync_copy(rows0, acc_sh.at[dst_v.at[j0]], add=True)
            pltpu.make_async_copy(g_hbm.at[src_v.at[j1]], rows1, gs1).wait()

            @pl.when(j1 + 1 < NCHUNK)
            def _():
                pltpu.async_copy(g_hbm.at[src_v.at[j1 + 1]], rows0, gs0)

            pltpu.sync_copy(rows1, acc_sh.at[dst_v.at[j1]], add=True)
            return carry

        lax.fori_loop(0, npair, body, 0)
        if NCHUNK % 2:
            j = NCHUNK - 1
            pltpu.make_async_copy(g_hbm.at[src_v.at[j]], rows0, gs0).wait()
            pltpu.sync_copy(rows0, acc_sh.at[dst_v.at[j]], add=True)
        plsc.subcore_barrier()
        pltpu.sync_copy(acc_sh.at[sl], out_hbm.at[cid, sl])

    return spmm


def _make_spmm64_spmem_kernel():
    D = 64

    @functools.partial(
        pl.kernel,
        mesh=plsc.VectorSubcoreMesh(**_MESH),
        compiler_params=pltpu.CompilerParams(use_tc_tiling_on_sc=False),
        out_type=jax.ShapeDtypeStruct((NC, N_ACC, D), jnp.float32),
        scratch_types=[
            pltpu.VMEM((NCHUNK, K), jnp.int32),
            pltpu.VMEM((NCHUNK, K), jnp.int32),
            pltpu.VMEM((K, D), jnp.float32),
            pltpu.VMEM((K, D), jnp.float32),
            pltpu.VMEM_SHARED((N, D), jnp.float32),
            pltpu.VMEM_SHARED((N_ACC, D), jnp.float32),
            pltpu.SemaphoreType.DMA,
            pltpu.SemaphoreType.DMA,
        ],
    )
    def spmm(g_hbm, src_hbm, dst_hbm, zeros_hbm, out_hbm,
             src_v, dst_v, rows0, rows1, g_sh, acc_sh, gs0, gs1):
        cid = lax.axis_index("c")
        sid = lax.axis_index("s")
        pltpu.sync_copy(src_hbm.at[cid, sid], src_v)
        pltpu.sync_copy(dst_hbm.at[cid, sid], dst_v)
        sl = pl.ds(sid * ROWS_PT, ROWS_PT)
        pltpu.sync_copy(zeros_hbm.at[sl], acc_sh.at[sl])

        @pl.when(sid < 10)
        def _():
            gl = pl.ds(sid * 1000, 1000)
            pltpu.sync_copy(g_hbm.at[gl], g_sh.at[gl])

        plsc.subcore_barrier()

        pltpu.async_copy(g_sh.at[src_v.at[0]], rows0, gs0)
        npair = NCHUNK // 2

        def body(i, carry):
            j0 = 2 * i
            j1 = 2 * i + 1
            pltpu.make_async_copy(g_sh.at[src_v.at[j0]], rows0, gs0).wait()
            pltpu.async_copy(g_sh.at[src_v.at[j1]], rows1, gs1)
            pltpu.sync_copy(rows0, acc_sh.at[dst_v.at[j0]], add=True)
            pltpu.make_async_copy(g_sh.at[src_v.at[j1]], rows1, gs1).wait()

            @pl.when(j1 + 1 < NCHUNK)
            def _():
                pltpu.async_copy(g_sh.at[src_v.at[j1 + 1]], rows0, gs0)

            pltpu.sync_copy(rows1, acc_sh.at[dst_v.at[j1]], add=True)
            return carry

        lax.fori_loop(0, npair, body, 0)
        if NCHUNK % 2:
            j = NCHUNK - 1
            pltpu.make_async_copy(g_sh.at[src_v.at[j]], rows0, gs0).wait()
            pltpu.sync_copy(rows0, acc_sh.at[dst_v.at[j]], add=True)
        plsc.subcore_barrier()
        pltpu.sync_copy(acc_sh.at[sl], out_hbm.at[cid, sl])

    return spmm


def _make_spmm1_kernel():
    @functools.partial(
        pl.kernel,
        mesh=plsc.VectorSubcoreMesh(**_MESH),
        compiler_params=pltpu.CompilerParams(use_tc_tiling_on_sc=False,
                                             needs_layout_passes=False),
        out_type=jax.ShapeDtypeStruct((NC, N_ACC, PAD_W), jnp.float32),
        scratch_types=[
            pltpu.VMEM((E_PT,), jnp.int32),
            pltpu.VMEM((NCHUNK, K), jnp.int32),
            pltpu.VMEM((N,), jnp.float32),
            pltpu.VMEM((K, PAD_W), jnp.float32),
            pltpu.VMEM_SHARED((N_ACC, PAD_W), jnp.float32),
        ],
    )
    def spmm1(g_hbm, src_hbm, dst_hbm, zeros_hbm, out_hbm,
              src_v, dst_v, g_v, rows_v, acc_sh):
        cid = lax.axis_index("c")
        sid = lax.axis_index("s")
        pltpu.sync_copy(src_hbm.at[cid, sid], src_v)
        pltpu.sync_copy(dst_hbm.at[cid, sid], dst_v)
        pltpu.sync_copy(g_hbm, g_v)
        pltpu.sync_copy(zeros_hbm.at[pl.ds(0, K)], rows_v)
        sl = pl.ds(sid * ROWS_PT, ROWS_PT)
        pltpu.sync_copy(zeros_hbm.at[sl], acc_sh.at[sl])
        plsc.subcore_barrier()

        iota = lax.iota(jnp.int32, 16)
        zcol = jnp.zeros((16,), jnp.int32)

        def body(j, carry):
            for g in range(K // 16):
                s = src_v[pl.ds(j * K + g * 16, 16)]
                v = plsc.load_gather(g_v, [s])
                plsc.store_scatter(rows_v, [iota + (g * 16), zcol], v)
            pltpu.sync_copy(rows_v, acc_sh.at[dst_v.at[j]], add=True)
            return carry

        lax.fori_loop(0, NCHUNK, body, 0)
        plsc.subcore_barrier()
        pltpu.sync_copy(acc_sh.at[sl], out_hbm.at[cid, sl])

    return spmm1


_deg_call = _make_degree_kernel()
_spmm128 = _make_spmm_kernel(128)
_spmm64 = _make_spmm64_spmem_kernel()
_spmm16 = _make_spmm1_kernel()


_TB = 1000
_GRID = (N // _TB,)


def _dis_block(degp):
    deg = degp[0, :, 0:1] + degp[1, :, 0:1]
    return jnp.where(deg > 0.0, lax.rsqrt(deg), 0.0)


def _t1a_body(x_ref, fcW_ref, fcb_ref, c1W_ref, x1_ref, h1_ref):
    xb = x_ref[...]
    x1_ref[...] = jnp.maximum(
        jnp.dot(xb, fcW_ref[...], preferred_element_type=jnp.float32) + fcb_ref[...], 0.0)
    h1_ref[...] = jnp.dot(xb, c1W_ref[...], preferred_element_type=jnp.float32)


def _t1b_body(h1_ref, degp_ref, g1_ref):
    g1_ref[...] = h1_ref[...] * _dis_block(degp_ref[...])


def _t2_body(x1_ref, acc1_ref, degp_ref, c1b_ref, fc1Wa_ref, fc1Wb_ref,
             fc1b_ref, c2Wa_ref, c2Wb_ref, x5_ref, g2_ref):
    dis = _dis_block(degp_ref[...])
    s1 = acc1_ref[0] + acc1_ref[1]
    x2 = jnp.maximum(dis * s1 + c1b_ref[...], 0.0)
    x1b = x1_ref[...]
    x5_ref[...] = jnp.maximum(
        jnp.dot(x1b, fc1Wa_ref[...], preferred_element_type=jnp.float32)
        + jnp.dot(x2, fc1Wb_ref[...], preferred_element_type=jnp.float32)
        + fc1b_ref[...], 0.0)
    g2_ref[...] = (jnp.dot(x1b, c2Wa_ref[...], preferred_element_type=jnp.float32)
                   + jnp.dot(x2, c2Wb_ref[...], preferred_element_type=jnp.float32)) * dis


def _t3_body(x5_ref, acc2_ref, degp_ref, c2b_ref, W9_ref, b9_ref, W3_ref,
             x9_ref, g3_ref):
    dis = _dis_block(degp_ref[...])
    s2 = acc2_ref[0] + acc2_ref[1]
    x6 = jnp.maximum(dis * s2 + c2b_ref[...], 0.0)
    x7 = x5_ref[...] + x6
    x9_ref[...] = jnp.dot(x7, W9_ref[...], preferred_element_type=jnp.float32) + b9_ref[...]
    g3_ref[...] = jnp.dot(x7, W3_ref[...], preferred_element_type=jnp.float32) * dis


def _t4_body(x9_ref, acc3_ref, degp_ref, c3b_ref, out_ref):
    dis = _dis_block(degp_ref[...])
    s3 = acc3_ref[0] + acc3_ref[1]
    out_ref[...] = x9_ref[...] + dis * s3 + c3b_ref[...]


def _row_spec(w):
    return pl.BlockSpec((_TB, w), lambda i: (i, 0))


def _part_spec(w):
    return pl.BlockSpec((NC, _TB, w), lambda i: (0, i, 0))


def _full_spec(shape):
    nd = len(shape)
    return pl.BlockSpec(shape, lambda i: (0,) * nd)


def _sds(shape):
    return jax.ShapeDtypeStruct(shape, jnp.float32)


_t1a_call = pl.pallas_call(
    _t1a_body,
    grid=_GRID,
    in_specs=[_row_spec(128), _full_spec((128, 128)), _full_spec((1, 128)),
              _full_spec((128, 128))],
    out_specs=[_row_spec(128), _row_spec(128)],
    out_shape=[_sds((N, 128)), _sds((N, 128))],
)

_t1b_call = pl.pallas_call(
    _t1b_body,
    grid=_GRID,
    in_specs=[_row_spec(128), _part_spec(PAD_W)],
    out_specs=_row_spec(128),
    out_shape=_sds((N, 128)),
)

_t2_call = pl.pallas_call(
    _t2_body,
    grid=_GRID,
    in_specs=[_row_spec(128), _part_spec(128), _part_spec(PAD_W),
              _full_spec((1, 128)), _full_spec((128, 64)), _full_spec((128, 64)),
              _full_spec((1, 64)), _full_spec((128, 64)), _full_spec((128, 64))],
    out_specs=[_row_spec(64), _row_spec(64)],
    out_shape=[_sds((N, 64)), _sds((N, 64))],
)

_t3_call = pl.pallas_call(
    _t3_body,
    grid=_GRID,
    in_specs=[_row_spec(64), _part_spec(64), _part_spec(PAD_W),
              _full_spec((1, 64)), _full_spec((64, PAD_W)),
              _full_spec((1, PAD_W)), _full_spec((64, 1))],
    out_specs=[_row_spec(PAD_W), _row_spec(1)],
    out_shape=[_sds((N, PAD_W)), _sds((N, 1))],
)

_t4_call = pl.pallas_call(
    _t4_body,
    grid=_GRID,
    in_specs=[_row_spec(PAD_W), _part_spec(PAD_W), _part_spec(PAD_W),
              _full_spec((1, PAD_W))],
    out_specs=_row_spec(PAD_W),
    out_shape=_sds((N, PAD_W)),
)


def _pad16(a):
    return jnp.pad(a, ((0, 0), (0, PAD_W - a.shape[1])))


def kernel(x, edge_index, fc_W, fc_b, conv1_W, conv1_b, fc1_W, fc1_b,
           conv2_W, conv2_b, fc2_W, fc2_b, conv3_W, conv3_b):
    src_r = edge_index[0].reshape(NC, NS, NCHUNK, K)
    dst_r = edge_index[1].reshape(NC, NS, NCHUNK, K)
    src_f = edge_index[0].reshape(NC, NS, E_PT)

    ones = jnp.ones((K, PAD_W), jnp.float32)
    zeros16 = jnp.zeros((N_ACC, PAD_W), jnp.float32)
    zeros64 = jnp.zeros((N_ACC, 64), jnp.float32)
    zeros128 = jnp.zeros((N_ACC, 128), jnp.float32)

    degp = _deg_call(dst_r, ones, zeros16)

    x1, h1 = _t1a_call(x, fc_W, fc_b.reshape(1, -1), conv1_W)
    g1 = _t1b_call(h1, degp)
    acc1 = _spmm128(g1, src_r, dst_r, zeros128)

    x5, g2 = _t2_call(x1, acc1, degp, conv1_b.reshape(1, -1),
                      fc1_W[:128], fc1_W[128:], fc1_b.reshape(1, -1),
                      conv2_W[:128], conv2_W[128:])
    acc2 = _spmm64(g2, src_r, dst_r, zeros64)

    x9p, g3 = _t3_call(x5, acc2, degp, conv2_b.reshape(1, -1),
                       _pad16(fc2_W), _pad16(fc2_b.reshape(1, 1)), conv3_W)
    acc3 = _spmm16(g3.reshape(-1), src_f, dst_r, zeros16)

    outp = _t4_call(x9p, acc3, degp, _pad16(conv3_b.reshape(1, 1)))
    return outp[:, 0:1]


# speedup vs baseline: 2.5756x; 1.1659x over previous
import functools

import jax
import jax.numpy as jnp
from jax import lax
from jax.experimental import pallas as pl
from jax.experimental.pallas import tpu as pltpu
from jax.experimental.pallas import tpu_sc as plsc

N = 10000
E = 320000
NC = 2
NS = 16
NW = NC * NS
K = 80
NCHUNK = 125
E_PT = NCHUNK * K
N_ACC = 10240
ROWS_PT = N_ACC // NS
PAD_W = 16

_MESH = dict(core_axis_name="c", subcore_axis_name="s")


def _make_degree_kernel():
    @functools.partial(
        pl.kernel,
        mesh=plsc.VectorSubcoreMesh(**_MESH),
        compiler_params=pltpu.CompilerParams(use_tc_tiling_on_sc=False),
        out_type=jax.ShapeDtypeStruct((NC, N_ACC, PAD_W), jnp.float32),
        scratch_types=[
            pltpu.VMEM((NCHUNK, K), jnp.int32),
            pltpu.VMEM((K, PAD_W), jnp.float32),
            pltpu.VMEM_SHARED((N_ACC, PAD_W), jnp.float32),
        ],
    )
    def deg_kernel(dst_hbm, ones_hbm, zeros_hbm, out_hbm, dst_v, ones_v, acc_sh):
        cid = lax.axis_index("c")
        sid = lax.axis_index("s")
        pltpu.sync_copy(dst_hbm.at[cid, sid], dst_v)
        pltpu.sync_copy(ones_hbm, ones_v)
        sl = pl.ds(sid * ROWS_PT, ROWS_PT)
        pltpu.sync_copy(zeros_hbm.at[sl], acc_sh.at[sl])
        plsc.subcore_barrier()

        def body(j, carry):
            pltpu.sync_copy(ones_v, acc_sh.at[dst_v.at[j]], add=True)
            return carry

        lax.fori_loop(0, NCHUNK, body, 0)
        plsc.subcore_barrier()
        pltpu.sync_copy(acc_sh.at[sl], out_hbm.at[cid, sl])

    return deg_kernel


def _make_spmm_kernel(D):
    NB = 3

    @functools.partial(
        pl.kernel,
        mesh=plsc.VectorSubcoreMesh(**_MESH),
        compiler_params=pltpu.CompilerParams(use_tc_tiling_on_sc=False),
        out_type=jax.ShapeDtypeStruct((NC, N, D), jnp.float32),
        scratch_types=[
            pltpu.VMEM((NCHUNK, K), jnp.int32),
            pltpu.VMEM((NCHUNK, K), jnp.int32),
            pltpu.VMEM((K, D), jnp.float32),
            pltpu.VMEM((K, D), jnp.float32),
            pltpu.VMEM((K, D), jnp.float32),
            pltpu.VMEM_SHARED((N, D), jnp.float32),
            pltpu.SemaphoreType.DMA,
            pltpu.SemaphoreType.DMA,
            pltpu.SemaphoreType.DMA,
        ],
    )
    def spmm(g_hbm, src_hbm, dst_hbm, zeros_hbm, out_hbm,
             src_v, dst_v, rows0, rows1, rows2, acc_sh, gs0, gs1, gs2):
        cid = lax.axis_index("c")
        sid = lax.axis_index("s")
        rows = (rows0, rows1, rows2)
        sems = (gs0, gs1, gs2)
        pltpu.sync_copy(src_hbm.at[cid, sid], src_v)
        pltpu.sync_copy(dst_hbm.at[cid, sid], dst_v)
        sl = pl.ds(sid * 1000, 1000)

        @pl.when(sid < 10)
        def _():
            pltpu.sync_copy(zeros_hbm.at[sl], acc_sh.at[sl])

        plsc.subcore_barrier()

        for j in range(NB):
            pltpu.async_copy(g_hbm.at[src_v.at[j]], rows[j], sems[j])
        ntrip = NCHUNK // NB

        def body(i, carry):
            for b in range(NB):
                j = NB * i + b
                pltpu.make_async_copy(g_hbm.at[src_v.at[j]], rows[b], sems[b]).wait()
                pltpu.sync_copy(rows[b], acc_sh.at[dst_v.at[j]], add=True)

                @pl.when(j + NB < NCHUNK)
                def _():
                    pltpu.async_copy(g_hbm.at[src_v.at[j + NB]], rows[b], sems[b])

            return carry

        lax.fori_loop(0, ntrip, body, 0)
        for j in range(NB * ntrip, NCHUNK):
            b = j % NB
            pltpu.make_async_copy(g_hbm.at[src_v.at[j]], rows[b], sems[b]).wait()
            pltpu.sync_copy(rows[b], acc_sh.at[dst_v.at[j]], add=True)
        plsc.subcore_barrier()

        @pl.when(sid < 10)
        def _():
            pltpu.sync_copy(acc_sh.at[sl], out_hbm.at[cid, sl])

    return spmm


def _make_spmm64_spmem_kernel():
    D = 64

    @functools.partial(
        pl.kernel,
        mesh=plsc.VectorSubcoreMesh(**_MESH),
        compiler_params=pltpu.CompilerParams(use_tc_tiling_on_sc=False),
        out_type=jax.ShapeDtypeStruct((NC, N_ACC, D), jnp.float32),
        scratch_types=[
            pltpu.VMEM((NCHUNK, K), jnp.int32),
            pltpu.VMEM((NCHUNK, K), jnp.int32),
            pltpu.VMEM((K, D), jnp.float32),
            pltpu.VMEM((K, D), jnp.float32),
            pltpu.VMEM_SHARED((N, D), jnp.float32),
            pltpu.VMEM_SHARED((N_ACC, D), jnp.float32),
            pltpu.SemaphoreType.DMA,
            pltpu.SemaphoreType.DMA,
        ],
    )
    def spmm(g_hbm, src_hbm, dst_hbm, zeros_hbm, out_hbm,
             src_v, dst_v, rows0, rows1, g_sh, acc_sh, gs0, gs1):
        cid = lax.axis_index("c")
        sid = lax.axis_index("s")
        pltpu.sync_copy(src_hbm.at[cid, sid], src_v)
        pltpu.sync_copy(dst_hbm.at[cid, sid], dst_v)
        sl = pl.ds(sid * ROWS_PT, ROWS_PT)
        pltpu.sync_copy(zeros_hbm.at[sl], acc_sh.at[sl])

        @pl.when(sid < 10)
        def _():
            gl = pl.ds(sid * 1000, 1000)
            pltpu.sync_copy(g_hbm.at[gl], g_sh.at[gl])

        plsc.subcore_barrier()

        pltpu.async_copy(g_sh.at[src_v.at[0]], rows0, gs0)
        npair = NCHUNK // 2

        def body(i, carry):
            j0 = 2 * i
            j1 = 2 * i + 1
            pltpu.make_async_copy(g_sh.at[src_v.at[j0]], rows0, gs0).wait()
            pltpu.async_copy(g_sh.at[src_v.at[j1]], rows1, gs1)
            pltpu.sync_copy(rows0, acc_sh.at[dst_v.at[j0]], add=True)
            pltpu.make_async_copy(g_sh.at[src_v.at[j1]], rows1, gs1).wait()

            @pl.when(j1 + 1 < NCHUNK)
            def _():
                pltpu.async_copy(g_sh.at[src_v.at[j1 + 1]], rows0, gs0)

            pltpu.sync_copy(rows1, acc_sh.at[dst_v.at[j1]], add=True)
            return carry

        lax.fori_loop(0, npair, body, 0)
        if NCHUNK % 2:
            j = NCHUNK - 1
            pltpu.make_async_copy(g_sh.at[src_v.at[j]], rows0, gs0).wait()
            pltpu.sync_copy(rows0, acc_sh.at[dst_v.at[j]], add=True)
        plsc.subcore_barrier()
        pltpu.sync_copy(acc_sh.at[sl], out_hbm.at[cid, sl])

    return spmm


def _make_spmm1_kernel():
    @functools.partial(
        pl.kernel,
        mesh=plsc.VectorSubcoreMesh(**_MESH),
        compiler_params=pltpu.CompilerParams(use_tc_tiling_on_sc=False,
                                             needs_layout_passes=False),
        out_type=jax.ShapeDtypeStruct((NC, N_ACC, PAD_W), jnp.float32),
        scratch_types=[
            pltpu.VMEM((E_PT,), jnp.int32),
            pltpu.VMEM((NCHUNK, K), jnp.int32),
            pltpu.VMEM((N,), jnp.float32),
            pltpu.VMEM((K, PAD_W), jnp.float32),
            pltpu.VMEM_SHARED((N_ACC, PAD_W), jnp.float32),
        ],
    )
    def spmm1(g_hbm, src_hbm, dst_hbm, zeros_hbm, out_hbm,
              src_v, dst_v, g_v, rows_v, acc_sh):
        cid = lax.axis_index("c")
        sid = lax.axis_index("s")
        pltpu.sync_copy(src_hbm.at[cid, sid], src_v)
        pltpu.sync_copy(dst_hbm.at[cid, sid], dst_v)
        pltpu.sync_copy(g_hbm, g_v)
        pltpu.sync_copy(zeros_hbm.at[pl.ds(0, K)], rows_v)
        sl = pl.ds(sid * ROWS_PT, ROWS_PT)
        pltpu.sync_copy(zeros_hbm.at[sl], acc_sh.at[sl])
        plsc.subcore_barrier()

        iota = lax.iota(jnp.int32, 16)
        zcol = jnp.zeros((16,), jnp.int32)

        def body(j, carry):
            for g in range(K // 16):
                s = src_v[pl.ds(j * K + g * 16, 16)]
                v = plsc.load_gather(g_v, [s])
                plsc.store_scatter(rows_v, [iota + (g * 16), zcol], v)
            pltpu.sync_copy(rows_v, acc_sh.at[dst_v.at[j]], add=True)
            return carry

        lax.fori_loop(0, NCHUNK, body, 0)
        plsc.subcore_barrier()
        pltpu.sync_copy(acc_sh.at[sl], out_hbm.at[cid, sl])

    return spmm1


_deg_call = _make_degree_kernel()
_spmm128 = _make_spmm_kernel(128)
_spmm64 = _make_spmm64_spmem_kernel()
_spmm16 = _make_spmm1_kernel()


_TB = 1000
_GRID = (N // _TB,)


def _dis_block(degp):
    deg = degp[0, :, 0:1] + degp[1, :, 0:1]
    return jnp.where(deg > 0.0, lax.rsqrt(deg), 0.0)


def _t1a_body(x_ref, fcW_ref, fcb_ref, c1W_ref, x1_ref, h1_ref):
    xb = x_ref[...]
    x1_ref[...] = jnp.maximum(
        jnp.dot(xb, fcW_ref[...], preferred_element_type=jnp.float32) + fcb_ref[...], 0.0)
    h1_ref[...] = jnp.dot(xb, c1W_ref[...], preferred_element_type=jnp.float32)


def _t1b_body(h1_ref, degp_ref, g1_ref):
    g1_ref[...] = h1_ref[...] * _dis_block(degp_ref[...])


def _t2_body(x1_ref, acc1_ref, degp_ref, c1b_ref, fc1Wa_ref, fc1Wb_ref,
             fc1b_ref, c2Wa_ref, c2Wb_ref, x5_ref, g2_ref):
    dis = _dis_block(degp_ref[...])
    s1 = acc1_ref[0] + acc1_ref[1]
    x2 = jnp.maximum(dis * s1 + c1b_ref[...], 0.0)
    x1b = x1_ref[...]
    x5_ref[...] = jnp.maximum(
        jnp.dot(x1b, fc1Wa_ref[...], preferred_element_type=jnp.float32)
        + jnp.dot(x2, fc1Wb_ref[...], preferred_element_type=jnp.float32)
        + fc1b_ref[...], 0.0)
    g2_ref[...] = (jnp.dot(x1b, c2Wa_ref[...], preferred_element_type=jnp.float32)
                   + jnp.dot(x2, c2Wb_ref[...], preferred_element_type=jnp.float32)) * dis


def _t3_body(x5_ref, acc2_ref, degp_ref, c2b_ref, W9_ref, b9_ref, W3_ref,
             x9_ref, g3_ref):
    dis = _dis_block(degp_ref[...])
    s2 = acc2_ref[0] + acc2_ref[1]
    x6 = jnp.maximum(dis * s2 + c2b_ref[...], 0.0)
    x7 = x5_ref[...] + x6
    x9_ref[...] = jnp.dot(x7, W9_ref[...], preferred_element_type=jnp.float32) + b9_ref[...]
    g3_ref[...] = jnp.dot(x7, W3_ref[...], preferred_element_type=jnp.float32) * dis


def _t4_body(x9_ref, acc3_ref, degp_ref, c3b_ref, out_ref):
    dis = _dis_block(degp_ref[...])
    s3 = acc3_ref[0] + acc3_ref[1]
    out_ref[...] = x9_ref[...] + dis * s3 + c3b_ref[...]


def _row_spec(w):
    return pl.BlockSpec((_TB, w), lambda i: (i, 0))


def _part_spec(w):
    return pl.BlockSpec((NC, _TB, w), lambda i: (0, i, 0))


def _full_spec(shape):
    nd = len(shape)
    return pl.BlockSpec(shape, lambda i: (0,) * nd)


def _sds(shape):
    return jax.ShapeDtypeStruct(shape, jnp.float32)


_t1a_call = pl.pallas_call(
    _t1a_body,
    grid=_GRID,
    in_specs=[_row_spec(128), _full_spec((128, 128)), _full_spec((1, 128)),
              _full_spec((128, 128))],
    out_specs=[_row_spec(128), _row_spec(128)],
    out_shape=[_sds((N, 128)), _sds((N, 128))],
)

_t1b_call = pl.pallas_call(
    _t1b_body,
    grid=_GRID,
    in_specs=[_row_spec(128), _part_spec(PAD_W)],
    out_specs=_row_spec(128),
    out_shape=_sds((N, 128)),
)

_t2_call = pl.pallas_call(
    _t2_body,
    grid=_GRID,
    in_specs=[_row_spec(128), _part_spec(128), _part_spec(PAD_W),
              _full_spec((1, 128)), _full_spec((128, 64)), _full_spec((128, 64)),
              _full_spec((1, 64)), _full_spec((128, 64)), _full_spec((128, 64))],
    out_specs=[_row_spec(64), _row_spec(64)],
    out_shape=[_sds((N, 64)), _sds((N, 64))],
)

_t3_call = pl.pallas_call(
    _t3_body,
    grid=_GRID,
    in_specs=[_row_spec(64), _part_spec(64), _part_spec(PAD_W),
              _full_spec((1, 64)), _full_spec((64, PAD_W)),
              _full_spec((1, PAD_W)), _full_spec((64, 1))],
    out_specs=[_row_spec(PAD_W), _row_spec(1)],
    out_shape=[_sds((N, PAD_W)), _sds((N, 1))],
)

_t4_call = pl.pallas_call(
    _t4_body,
    grid=_GRID,
    in_specs=[_row_spec(PAD_W), _part_spec(PAD_W), _part_spec(PAD_W),
              _full_spec((1, PAD_W))],
    out_specs=_row_spec(PAD_W),
    out_shape=_sds((N, PAD_W)),
)


def _pad16(a):
    return jnp.pad(a, ((0, 0), (0, PAD_W - a.shape[1])))


def kernel(x, edge_index, fc_W, fc_b, conv1_W, conv1_b, fc1_W, fc1_b,
           conv2_W, conv2_b, fc2_W, fc2_b, conv3_W, conv3_b):
    src_r = edge_index[0].reshape(NC, NS, NCHUNK, K)
    dst_r = edge_index[1].reshape(NC, NS, NCHUNK, K)
    src_f = edge_index[0].reshape(NC, NS, E_PT)

    ones = jnp.ones((K, PAD_W), jnp.float32)
    zeros16 = jnp.zeros((N_ACC, PAD_W), jnp.float32)
    zeros64 = jnp.zeros((N_ACC, 64), jnp.float32)
    zeros128 = jnp.zeros((N_ACC, 128), jnp.float32)

    degp = _deg_call(dst_r, ones, zeros16)

    x1, h1 = _t1a_call(x, fc_W, fc_b.reshape(1, -1), conv1_W)
    g1 = _t1b_call(h1, degp)
    acc1 = _spmm128(g1, src_r, dst_r, zeros128)

    x5, g2 = _t2_call(x1, acc1, degp, conv1_b.reshape(1, -1),
                      fc1_W[:128], fc1_W[128:], fc1_b.reshape(1, -1),
                      conv2_W[:128], conv2_W[128:])
    acc2 = _spmm64(g2, src_r, dst_r, zeros64)

    x9p, g3 = _t3_call(x5, acc2, degp, conv2_b.reshape(1, -1),
                       _pad16(fc2_W), _pad16(fc2_b.reshape(1, 1)), conv3_W)
    acc3 = _spmm16(g3.reshape(-1), src_f, dst_r, zeros16)

    outp = _t4_call(x9p, acc3, degp, _pad16(conv3_b.reshape(1, 1)))
    return outp[:, 0:1]


# speedup vs baseline: 2.5999x; 1.0094x over previous
import functools

import jax
import jax.numpy as jnp
from jax import lax
from jax.experimental import pallas as pl
from jax.experimental.pallas import tpu as pltpu
from jax.experimental.pallas import tpu_sc as plsc

N = 10000
E = 320000
NC = 2
NS = 16
NW = NC * NS
K = 80
NCHUNK = 125
E_PT = NCHUNK * K
N_ACC = 10240
ROWS_PT = N_ACC // NS
PAD_W = 16

_MESH = dict(core_axis_name="c", subcore_axis_name="s")


def _make_degree_kernel():
    @functools.partial(
        pl.kernel,
        mesh=plsc.VectorSubcoreMesh(**_MESH),
        compiler_params=pltpu.CompilerParams(use_tc_tiling_on_sc=False),
        out_type=jax.ShapeDtypeStruct((NC, N_ACC, PAD_W), jnp.float32),
        scratch_types=[
            pltpu.VMEM((NCHUNK, K), jnp.int32),
            pltpu.VMEM((K, PAD_W), jnp.float32),
            pltpu.VMEM_SHARED((N_ACC, PAD_W), jnp.float32),
        ],
    )
    def deg_kernel(dst_hbm, ones_hbm, zeros_hbm, out_hbm, dst_v, ones_v, acc_sh):
        cid = lax.axis_index("c")
        sid = lax.axis_index("s")
        pltpu.sync_copy(dst_hbm.at[cid, sid], dst_v)
        pltpu.sync_copy(ones_hbm, ones_v)
        sl = pl.ds(sid * ROWS_PT, ROWS_PT)
        pltpu.sync_copy(zeros_hbm.at[sl], acc_sh.at[sl])
        plsc.subcore_barrier()

        def body(j, carry):
            pltpu.sync_copy(ones_v, acc_sh.at[dst_v.at[j]], add=True)
            return carry

        lax.fori_loop(0, NCHUNK, body, 0)
        plsc.subcore_barrier()
        pltpu.sync_copy(acc_sh.at[sl], out_hbm.at[cid, sl])

    return deg_kernel


def _make_spmm_kernel(D):
    NB = 3

    @functools.partial(
        pl.kernel,
        mesh=plsc.VectorSubcoreMesh(**_MESH),
        compiler_params=pltpu.CompilerParams(use_tc_tiling_on_sc=False),
        out_type=jax.ShapeDtypeStruct((NC, N, D), jnp.float32),
        scratch_types=[
            pltpu.VMEM((NCHUNK, K), jnp.int32),
            pltpu.VMEM((NCHUNK, K), jnp.int32),
            pltpu.VMEM((K, D), jnp.float32),
            pltpu.VMEM((K, D), jnp.float32),
            pltpu.VMEM((K, D), jnp.float32),
            pltpu.VMEM_SHARED((N, D), jnp.float32),
            pltpu.SemaphoreType.DMA,
            pltpu.SemaphoreType.DMA,
            pltpu.SemaphoreType.DMA,
        ],
    )
    def spmm(g_hbm, src_hbm, dst_hbm, zeros_hbm, out_hbm,
             src_v, dst_v, rows0, rows1, rows2, acc_sh, gs0, gs1, gs2):
        cid = lax.axis_index("c")
        sid = lax.axis_index("s")
        rows = (rows0, rows1, rows2)
        sems = (gs0, gs1, gs2)
        pltpu.sync_copy(src_hbm.at[cid, sid], src_v)
        pltpu.sync_copy(dst_hbm.at[cid, sid], dst_v)
        sl = pl.ds(sid * 1000, 1000)

        @pl.when(sid < 10)
        def _():
            pltpu.sync_copy(zeros_hbm.at[sl], acc_sh.at[sl])

        plsc.subcore_barrier()

        for j in range(NB):
            pltpu.async_copy(g_hbm.at[src_v.at[j]], rows[j], sems[j])
        ntrip = NCHUNK // NB

        def body(i, carry):
            for b in range(NB):
                j = NB * i + b
                pltpu.make_async_copy(g_hbm.at[src_v.at[j]], rows[b], sems[b]).wait()
                pltpu.sync_copy(rows[b], acc_sh.at[dst_v.at[j]], add=True)

                @pl.when(j + NB < NCHUNK)
                def _():
                    pltpu.async_copy(g_hbm.at[src_v.at[j + NB]], rows[b], sems[b])

            return carry

        lax.fori_loop(0, ntrip, body, 0)
        for j in range(NB * ntrip, NCHUNK):
            b = j % NB
            pltpu.make_async_copy(g_hbm.at[src_v.at[j]], rows[b], sems[b]).wait()
            pltpu.sync_copy(rows[b], acc_sh.at[dst_v.at[j]], add=True)
        plsc.subcore_barrier()

        @pl.when(sid < 10)
        def _():
            pltpu.sync_copy(acc_sh.at[sl], out_hbm.at[cid, sl])

    return spmm


def _make_spmm64_spmem_kernel():
    D = 64

    @functools.partial(
        pl.kernel,
        mesh=plsc.VectorSubcoreMesh(**_MESH),
        compiler_params=pltpu.CompilerParams(use_tc_tiling_on_sc=False),
        out_type=jax.ShapeDtypeStruct((NC, N_ACC, D), jnp.float32),
        scratch_types=[
            pltpu.VMEM((NCHUNK, K), jnp.int32),
            pltpu.VMEM((NCHUNK, K), jnp.int32),
            pltpu.VMEM((K, D), jnp.float32),
            pltpu.VMEM((K, D), jnp.float32),
            pltpu.VMEM((K, D), jnp.float32),
            pltpu.VMEM_SHARED((N, D), jnp.float32),
            pltpu.VMEM_SHARED((N_ACC, D), jnp.float32),
            pltpu.SemaphoreType.DMA,
            pltpu.SemaphoreType.DMA,
            pltpu.SemaphoreType.DMA,
        ],
    )
    def spmm(g_hbm, src_hbm, dst_hbm, zeros_hbm, out_hbm,
             src_v, dst_v, rows0, rows1, rows2, g_sh, acc_sh, gs0, gs1, gs2):
        NB = 3
        cid = lax.axis_index("c")
        sid = lax.axis_index("s")
        rows = (rows0, rows1, rows2)
        sems = (gs0, gs1, gs2)
        pltpu.sync_copy(src_hbm.at[cid, sid], src_v)
        pltpu.sync_copy(dst_hbm.at[cid, sid], dst_v)
        sl = pl.ds(sid * ROWS_PT, ROWS_PT)
        pltpu.sync_copy(zeros_hbm.at[sl], acc_sh.at[sl])

        @pl.when(sid < 10)
        def _():
            gl = pl.ds(sid * 1000, 1000)
            pltpu.sync_copy(g_hbm.at[gl], g_sh.at[gl])

        plsc.subcore_barrier()

        for j in range(NB):
            pltpu.async_copy(g_sh.at[src_v.at[j]], rows[j], sems[j])
        ntrip = NCHUNK // NB

        def body(i, carry):
            for b in range(NB):
                j = NB * i + b
                pltpu.make_async_copy(g_sh.at[src_v.at[j]], rows[b], sems[b]).wait()
                pltpu.sync_copy(rows[b], acc_sh.at[dst_v.at[j]], add=True)

                @pl.when(j + NB < NCHUNK)
                def _():
                    pltpu.async_copy(g_sh.at[src_v.at[j + NB]], rows[b], sems[b])

            return carry

        lax.fori_loop(0, ntrip, body, 0)
        for j in range(NB * ntrip, NCHUNK):
            b = j % NB
            pltpu.make_async_copy(g_sh.at[src_v.at[j]], rows[b], sems[b]).wait()
            pltpu.sync_copy(rows[b], acc_sh.at[dst_v.at[j]], add=True)
        plsc.subcore_barrier()
        pltpu.sync_copy(acc_sh.at[sl], out_hbm.at[cid, sl])

    return spmm


def _make_spmm1_kernel():
    @functools.partial(
        pl.kernel,
        mesh=plsc.VectorSubcoreMesh(**_MESH),
        compiler_params=pltpu.CompilerParams(use_tc_tiling_on_sc=False,
                                             needs_layout_passes=False),
        out_type=jax.ShapeDtypeStruct((NC, N_ACC, PAD_W), jnp.float32),
        scratch_types=[
            pltpu.VMEM((E_PT,), jnp.int32),
            pltpu.VMEM((NCHUNK, K), jnp.int32),
            pltpu.VMEM((N,), jnp.float32),
            pltpu.VMEM((K, PAD_W), jnp.float32),
            pltpu.VMEM_SHARED((N_ACC, PAD_W), jnp.float32),
        ],
    )
    def spmm1(g_hbm, src_hbm, dst_hbm, zeros_hbm, out_hbm,
              src_v, dst_v, g_v, rows_v, acc_sh):
        cid = lax.axis_index("c")
        sid = lax.axis_index("s")
        pltpu.sync_copy(src_hbm.at[cid, sid], src_v)
        pltpu.sync_copy(dst_hbm.at[cid, sid], dst_v)
        pltpu.sync_copy(g_hbm, g_v)
        pltpu.sync_copy(zeros_hbm.at[pl.ds(0, K)], rows_v)
        sl = pl.ds(sid * ROWS_PT, ROWS_PT)
        pltpu.sync_copy(zeros_hbm.at[sl], acc_sh.at[sl])
        plsc.subcore_barrier()

        iota = lax.iota(jnp.int32, 16)
        zcol = jnp.zeros((16,), jnp.int32)

        def body(j, carry):
            for g in range(K // 16):
                s = src_v[pl.ds(j * K + g * 16, 16)]
                v = plsc.load_gather(g_v, [s])
                plsc.store_scatter(rows_v, [iota + (g * 16), zcol], v)
            pltpu.sync_copy(rows_v, acc_sh.at[dst_v.at[j]], add=True)
            return carry

        lax.fori_loop(0, NCHUNK, body, 0)
        plsc.subcore_barrier()
        pltpu.sync_copy(acc_sh.at[sl], out_hbm.at[cid, sl])

    return spmm1


_deg_call = _make_degree_kernel()
_spmm128 = _make_spmm_kernel(128)
_spmm64 = _make_spmm64_spmem_kernel()
_spmm16 = _make_spmm1_kernel()


_TB = 1000
_GRID = (N // _TB,)


def _dis_block(degp):
    deg = degp[0, :, 0:1] + degp[1, :, 0:1]
    return jnp.where(deg > 0.0, lax.rsqrt(deg), 0.0)


def _t1a_body(x_ref, fcW_ref, fcb_ref, c1W_ref, x1_ref, h1_ref):
    xb = x_ref[...]
    x1_ref[...] = jnp.maximum(
        jnp.dot(xb, fcW_ref[...], preferred_element_type=jnp.float32) + fcb_ref[...], 0.0)
    h1_ref[...] = jnp.dot(xb, c1W_ref[...], preferred_element_type=jnp.float32)


def _t1b_body(h1_ref, degp_ref, g1_ref):
    g1_ref[...] = h1_ref[...] * _dis_block(degp_ref[...])


def _t2_body(x1_ref, acc1_ref, degp_ref, c1b_ref, fc1Wa_ref, fc1Wb_ref,
             fc1b_ref, c2Wa_ref, c2Wb_ref, x5_ref, g2_ref):
    dis = _dis_block(degp_ref[...])
    s1 = acc1_ref[0] + acc1_ref[1]
    x2 = jnp.maximum(dis * s1 + c1b_ref[...], 0.0)
    x1b = x1_ref[...]
    x5_ref[...] = jnp.maximum(
        jnp.dot(x1b, fc1Wa_ref[...], preferred_element_type=jnp.float32)
        + jnp.dot(x2, fc1Wb_ref[...], preferred_element_type=jnp.float32)
        + fc1b_ref[...], 0.0)
    g2_ref[...] = (jnp.dot(x1b, c2Wa_ref[...], preferred_element_type=jnp.float32)
                   + jnp.dot(x2, c2Wb_ref[...], preferred_element_type=jnp.float32)) * dis


def _t3_body(x5_ref, acc2_ref, degp_ref, c2b_ref, W9_ref, b9_ref, W3_ref,
             x9_ref, g3_ref):
    dis = _dis_block(degp_ref[...])
    s2 = acc2_ref[0] + acc2_ref[1]
    x6 = jnp.maximum(dis * s2 + c2b_ref[...], 0.0)
    x7 = x5_ref[...] + x6
    x9_ref[...] = jnp.dot(x7, W9_ref[...], preferred_element_type=jnp.float32) + b9_ref[...]
    g3_ref[...] = jnp.dot(x7, W3_ref[...], preferred_element_type=jnp.float32) * dis


def _t4_body(x9_ref, acc3_ref, degp_ref, c3b_ref, out_ref):
    dis = _dis_block(degp_ref[...])
    s3 = acc3_ref[0] + acc3_ref[1]
    out_ref[...] = x9_ref[...] + dis * s3 + c3b_ref[...]


def _row_spec(w):
    return pl.BlockSpec((_TB, w), lambda i: (i, 0))


def _part_spec(w):
    return pl.BlockSpec((NC, _TB, w), lambda i: (0, i, 0))


def _full_spec(shape):
    nd = len(shape)
    return pl.BlockSpec(shape, lambda i: (0,) * nd)


def _sds(shape):
    return jax.ShapeDtypeStruct(shape, jnp.float32)


_t1a_call = pl.pallas_call(
    _t1a_body,
    grid=_GRID,
    in_specs=[_row_spec(128), _full_spec((128, 128)), _full_spec((1, 128)),
              _full_spec((128, 128))],
    out_specs=[_row_spec(128), _row_spec(128)],
    out_shape=[_sds((N, 128)), _sds((N, 128))],
)

_t1b_call = pl.pallas_call(
    _t1b_body,
    grid=_GRID,
    in_specs=[_row_spec(128), _part_spec(PAD_W)],
    out_specs=_row_spec(128),
    out_shape=_sds((N, 128)),
)

_t2_call = pl.pallas_call(
    _t2_body,
    grid=_GRID,
    in_specs=[_row_spec(128), _part_spec(128), _part_spec(PAD_W),
              _full_spec((1, 128)), _full_spec((128, 64)), _full_spec((128, 64)),
              _full_spec((1, 64)), _full_spec((128, 64)), _full_spec((128, 64))],
    out_specs=[_row_spec(64), _row_spec(64)],
    out_shape=[_sds((N, 64)), _sds((N, 64))],
)

_t3_call = pl.pallas_call(
    _t3_body,
    grid=_GRID,
    in_specs=[_row_spec(64), _part_spec(64), _part_spec(PAD_W),
              _full_spec((1, 64)), _full_spec((64, PAD_W)),
              _full_spec((1, PAD_W)), _full_spec((64, 1))],
    out_specs=[_row_spec(PAD_W), _row_spec(1)],
    out_shape=[_sds((N, PAD_W)), _sds((N, 1))],
)

_t4_call = pl.pallas_call(
    _t4_body,
    grid=_GRID,
    in_specs=[_row_spec(PAD_W), _part_spec(PAD_W), _part_spec(PAD_W),
              _full_spec((1, PAD_W))],
    out_specs=_row_spec(PAD_W),
    out_shape=_sds((N, PAD_W)),
)


def _pad16(a):
    return jnp.pad(a, ((0, 0), (0, PAD_W - a.shape[1])))


def kernel(x, edge_index, fc_W, fc_b, conv1_W, conv1_b, fc1_W, fc1_b,
           conv2_W, conv2_b, fc2_W, fc2_b, conv3_W, conv3_b):
    src_r = edge_index[0].reshape(NC, NS, NCHUNK, K)
    dst_r = edge_index[1].reshape(NC, NS, NCHUNK, K)
    src_f = edge_index[0].reshape(NC, NS, E_PT)

    ones = jnp.ones((K, PAD_W), jnp.float32)
    zeros16 = jnp.zeros((N_ACC, PAD_W), jnp.float32)
    zeros64 = jnp.zeros((N_ACC, 64), jnp.float32)
    zeros128 = jnp.zeros((N_ACC, 128), jnp.float32)

    degp = _deg_call(dst_r, ones, zeros16)

    x1, h1 = _t1a_call(x, fc_W, fc_b.reshape(1, -1), conv1_W)
    g1 = _t1b_call(h1, degp)
    acc1 = _spmm128(g1, src_r, dst_r, zeros128)

    x5, g2 = _t2_call(x1, acc1, degp, conv1_b.reshape(1, -1),
                      fc1_W[:128], fc1_W[128:], fc1_b.reshape(1, -1),
                      conv2_W[:128], conv2_W[128:])
    acc2 = _spmm64(g2, src_r, dst_r, zeros64)

    x9p, g3 = _t3_call(x5, acc2, degp, conv2_b.reshape(1, -1),
                       _pad16(fc2_W), _pad16(fc2_b.reshape(1, 1)), conv3_W)
    acc3 = _spmm16(g3.reshape(-1), src_f, dst_r, zeros16)

    outp = _t4_call(x9p, acc3, degp, _pad16(conv3_b.reshape(1, 1)))
    return outp[:, 0:1]
